# Initial kernel scaffold; baseline (speedup 1.0000x reference)
import jax
import jax.numpy as jnp
from jax.experimental import pallas as pl


def kernel(features, edges, W1, b1, fc1_W, fc1_b, fc2_W, fc2_b):
    raise NotImplementedError("write your pallas kernel here")


# speedup vs baseline: 15.8165x; 15.8165x over previous
import functools

import jax
import jax.numpy as jnp
from jax import lax
from jax.experimental import pallas as pl
from jax.experimental.pallas import tpu as pltpu
from jax.experimental.pallas import tpu_sc as plsc

N = 10000
E = 320000
DF = 128
DG = 128
D1 = 64
D2 = 16

NC = 2
NS = 16
NW = NC * NS
EPW = E // NW
NPAD = 10240
SPW = NPAD // NS

CHUNK = 80
NCHUNK = EPW // CHUNK


def _mesh():
    return plsc.VectorSubcoreMesh(core_axis_name="c", subcore_axis_name="s")


def _zero_ref(ref, nrows, ncols16):
    def body(r, _):
        for c in range(ncols16):
            ref[r, pl.ds(c * 16, 16)] = jnp.zeros((16,), jnp.float32)
        return 0
    lax.fori_loop(0, nrows, body, 0, unroll=4)


@functools.cache
def _make_sc_degrees():
    @functools.partial(
        pl.kernel,
        mesh=_mesh(),
        compiler_params=pltpu.CompilerParams(use_tc_tiling_on_sc=False),
        out_type=(
            jax.ShapeDtypeStruct((NC, NPAD), jnp.float32),
            jax.ShapeDtypeStruct((NC, NPAD), jnp.float32),
        ),
        scratch_types=(
            pltpu.VMEM_SHARED((NPAD,), jnp.float32),
            pltpu.VMEM_SHARED((NPAD,), jnp.float32),
            pltpu.VMEM((CHUNK,), jnp.int32),
            pltpu.VMEM((CHUNK,), jnp.int32),
            pltpu.VMEM((CHUNK,), jnp.float32),
            pltpu.VMEM((SPW,), jnp.float32),
        ),
    )
    def sc_degrees(row_hbm, col_hbm, indeg_hbm, outdeg_hbm,
                   in_acc, out_acc, ridx, cidx, ones_v, zbuf):
        c = lax.axis_index("c")
        s = lax.axis_index("s")
        wid = c * NS + s
        for j in range(CHUNK // 16):
            ones_v[pl.ds(j * 16, 16)] = jnp.ones((16,), jnp.float32)
        for j in range(SPW // 16):
            zbuf[pl.ds(j * 16, 16)] = jnp.zeros((16,), jnp.float32)
        pltpu.sync_copy(zbuf, in_acc.at[pl.ds(s * SPW, SPW)])
        pltpu.sync_copy(zbuf, out_acc.at[pl.ds(s * SPW, SPW)])
        plsc.subcore_barrier()

        base = wid * EPW

        def body(j, _):
            off = base + j * CHUNK
            pltpu.sync_copy(row_hbm.at[pl.ds(off, CHUNK)], ridx)
            pltpu.sync_copy(col_hbm.at[pl.ds(off, CHUNK)], cidx)
            pltpu.sync_copy(ones_v, out_acc.at[ridx], add=True)
            pltpu.sync_copy(ones_v, in_acc.at[cidx], add=True)
            return 0

        lax.fori_loop(0, NCHUNK, body, 0)
        plsc.subcore_barrier()
        pltpu.sync_copy(in_acc.at[pl.ds(s * SPW, SPW)],
                        indeg_hbm.at[c, pl.ds(s * SPW, SPW)])
        pltpu.sync_copy(out_acc.at[pl.ds(s * SPW, SPW)],
                        outdeg_hbm.at[c, pl.ds(s * SPW, SPW)])

    return sc_degrees


def _sc_degrees(row, col):
    return _make_sc_degrees()(row, col)


ZROWS = SPW // 5


@functools.cache
def _make_sc_agg():
    @functools.partial(
        pl.kernel,
        mesh=_mesh(),
        out_type=jax.ShapeDtypeStruct((NC, NPAD, DG), jnp.float32),
        scratch_types=(
            pltpu.VMEM_SHARED((NPAD, DG), jnp.float32),
            pltpu.VMEM((CHUNK,), jnp.int32),
            pltpu.VMEM((CHUNK,), jnp.int32),
            pltpu.VMEM((CHUNK, DG), jnp.float32),
            pltpu.VMEM((ZROWS, DG), jnp.float32),
            pltpu.SemaphoreType.DMA,
        ),
    )
    def sc_agg(row_hbm, col_hbm, y_hbm, out_hbm,
               acc, ridx, cidx, rows_v, zbuf, sem):
        c = lax.axis_index("c")
        s = lax.axis_index("s")
        wid = c * NS + s
        _zero_ref(zbuf, ZROWS, DG // 16)
        for j in range(SPW // ZROWS):
            pltpu.sync_copy(
                zbuf, acc.at[pl.ds(s * SPW + j * ZROWS, ZROWS), :])
        plsc.subcore_barrier()

        base = wid * EPW

        def body(j, _):
            off = base + j * CHUNK
            pltpu.sync_copy(row_hbm.at[pl.ds(off, CHUNK)], ridx)
            pltpu.sync_copy(col_hbm.at[pl.ds(off, CHUNK)], cidx)
            pltpu.async_copy(y_hbm.at[ridx], rows_v, sem).wait()
            pltpu.sync_copy(rows_v, acc.at[cidx], add=True)
            return 0

        lax.fori_loop(0, NCHUNK, body, 0)
        plsc.subcore_barrier()
        pltpu.sync_copy(acc.at[pl.ds(s * SPW, SPW), :],
                        out_hbm.at[c, pl.ds(s * SPW, SPW), :])

    return sc_agg


def _sc_agg(row, col, Y):
    return _make_sc_agg()(row, col, Y)


@functools.cache
def _make_sc_lap():
    @functools.partial(
        pl.kernel,
        mesh=_mesh(),
        compiler_params=pltpu.CompilerParams(use_tc_tiling_on_sc=False),
        out_type=jax.ShapeDtypeStruct((NC, NPAD, D2), jnp.float32),
        scratch_types=(
            pltpu.VMEM_SHARED((NPAD, D2), jnp.float32),
            pltpu.VMEM((CHUNK,), jnp.int32),
            pltpu.VMEM((CHUNK,), jnp.int32),
            pltpu.VMEM((CHUNK, D2), jnp.float32),
            pltpu.VMEM((SPW, D2), jnp.float32),
            pltpu.SemaphoreType.DMA,
        ),
    )
    def sc_lap(row_hbm, col_hbm, u_hbm, out_hbm,
               acc, ridx, cidx, rows_v, zbuf, sem):
        c = lax.axis_index("c")
        s = lax.axis_index("s")
        wid = c * NS + s
        _zero_ref(zbuf, SPW, 1)
        pltpu.sync_copy(zbuf, acc.at[pl.ds(s * SPW, SPW), :])
        plsc.subcore_barrier()

        base = wid * EPW

        def body(j, _):
            off = base + j * CHUNK
            pltpu.sync_copy(col_hbm.at[pl.ds(off, CHUNK)], cidx)
            pltpu.sync_copy(row_hbm.at[pl.ds(off, CHUNK)], ridx)
            pltpu.async_copy(u_hbm.at[cidx], rows_v, sem).wait()
            pltpu.sync_copy(rows_v, acc.at[ridx], add=True)
            return 0

        lax.fori_loop(0, NCHUNK, body, 0)
        plsc.subcore_barrier()
        pltpu.sync_copy(acc.at[pl.ds(s * SPW, SPW), :],
                        out_hbm.at[c, pl.ds(s * SPW, SPW), :])

    return sc_lap


def _sc_lap(row, col, U):
    return _make_sc_lap()(row, col, U)


RB = 1000


def _tca_body(feat, w1, indeg0, indeg1, outdeg0, outdeg1,
              xw_o, y_o, dinv_o, dinvl_o):
    xw = jnp.dot(feat[...], w1[...], preferred_element_type=jnp.float32)
    indeg = indeg0[...] + indeg1[...]
    outdeg = outdeg0[...] + outdeg1[...]
    dinv = lax.rsqrt(indeg + 1.0)
    dinvl = jnp.where(outdeg > 0, lax.rsqrt(jnp.maximum(outdeg, 1.0)), 0.0)
    xw_o[...] = xw
    y_o[...] = dinv * xw
    dinv_o[...] = dinv
    dinvl_o[...] = dinvl


def _tc_a(features, W1, indeg0, indeg1, outdeg0, outdeg1):
    g = N // RB
    deg_spec = pl.BlockSpec((RB, 1), lambda i: (i, 0))
    return pl.pallas_call(
        _tca_body,
        grid=(g,),
        in_specs=[
            pl.BlockSpec((RB, DF), lambda i: (i, 0)),
            pl.BlockSpec((DF, DG), lambda i: (0, 0)),
            deg_spec, deg_spec, deg_spec, deg_spec,
        ],
        out_specs=[
            pl.BlockSpec((RB, DG), lambda i: (i, 0)),
            pl.BlockSpec((RB, DG), lambda i: (i, 0)),
            deg_spec, deg_spec,
        ],
        out_shape=[
            jax.ShapeDtypeStruct((N, DG), jnp.float32),
            jax.ShapeDtypeStruct((N, DG), jnp.float32),
            jax.ShapeDtypeStruct((N, 1), jnp.float32),
            jax.ShapeDtypeStruct((N, 1), jnp.float32),
        ],
    )(features, W1, indeg0, indeg1, outdeg0, outdeg1)


def _tcb_body(agg0, agg1, xw, dinv, dinvl, b1, w1, bb1, w2, bb2,
              s_o, u_o, cs_o, sts_o):
    di = dinv[...]
    agg = agg0[...] + agg1[...]
    nf2 = di * agg + (di * di) * xw[...] + b1[...]
    h1 = jnp.tanh(jnp.dot(nf2, w1[...], preferred_element_type=jnp.float32)
                  + bb1[...])
    lg = jnp.dot(h1, w2[...], preferred_element_type=jnp.float32) + bb2[...]
    m = jnp.max(lg, axis=1, keepdims=True)
    ex = jnp.exp(lg - m)
    S = ex / jnp.sum(ex, axis=1, keepdims=True)
    s_o[...] = S
    u_o[...] = dinvl[...] * S

    @pl.when(pl.program_id(0) == 0)
    def _():
        cs_o[...] = jnp.zeros_like(cs_o)
        sts_o[...] = jnp.zeros_like(sts_o)

    cs_o[...] += jnp.sum(nf2, axis=0, keepdims=True)
    sts_o[...] += lax.dot_general(S, S, (((0,), (0,)), ((), ())),
                                  preferred_element_type=jnp.float32)


def _tc_b(agg0, agg1, xw, dinv, dinvl, b1, fc1_W, fc1_b, fc2_W, fc2_b):
    g = N // RB
    deg_spec = pl.BlockSpec((RB, 1), lambda i: (i, 0))
    return pl.pallas_call(
        _tcb_body,
        grid=(g,),
        in_specs=[
            pl.BlockSpec((RB, DG), lambda i: (i, 0)),
            pl.BlockSpec((RB, DG), lambda i: (i, 0)),
            pl.BlockSpec((RB, DG), lambda i: (i, 0)),
            deg_spec, deg_spec,
            pl.BlockSpec((1, DG), lambda i: (0, 0)),
            pl.BlockSpec((DG, D1), lambda i: (0, 0)),
            pl.BlockSpec((1, D1), lambda i: (0, 0)),
            pl.BlockSpec((D1, D2), lambda i: (0, 0)),
            pl.BlockSpec((1, D2), lambda i: (0, 0)),
        ],
        out_specs=[
            pl.BlockSpec((RB, D2), lambda i: (i, 0)),
            pl.BlockSpec((RB, D2), lambda i: (i, 0)),
            pl.BlockSpec((1, DG), lambda i: (0, 0)),
            pl.BlockSpec((D2, D2), lambda i: (0, 0)),
        ],
        out_shape=[
            jax.ShapeDtypeStruct((N, D2), jnp.float32),
            jax.ShapeDtypeStruct((N, D2), jnp.float32),
            jax.ShapeDtypeStruct((1, DG), jnp.float32),
            jax.ShapeDtypeStruct((D2, D2), jnp.float32),
        ],
    )(agg0, agg1, xw, dinv, dinvl, b1, fc1_W, fc1_b, fc2_W, fc2_b)


def _tcc_body(u, t0, t1, utt_o):
    t = t0[...] + t1[...]

    @pl.when(pl.program_id(0) == 0)
    def _():
        utt_o[...] = jnp.zeros_like(utt_o)

    utt_o[...] += lax.dot_general(u[...], t, (((0,), (0,)), ((), ())),
                                  preferred_element_type=jnp.float32)


def _tc_c(U, t0, t1):
    g = N // RB
    return pl.pallas_call(
        _tcc_body,
        grid=(g,),
        in_specs=[
            pl.BlockSpec((RB, D2), lambda i: (i, 0)),
            pl.BlockSpec((RB, D2), lambda i: (i, 0)),
            pl.BlockSpec((RB, D2), lambda i: (i, 0)),
        ],
        out_specs=[pl.BlockSpec((D2, D2), lambda i: (0, 0))],
        out_shape=[jax.ShapeDtypeStruct((D2, D2), jnp.float32)],
    )(U, t0, t1)


def kernel(features, edges, W1, b1, fc1_W, fc1_b, fc2_W, fc2_b):
    row = edges[0]
    col = edges[1]

    indeg_p, outdeg_p = _sc_degrees(row, col)

    xw, Y, dinv, dinvl = _tc_a(
        features, W1,
        indeg_p[0, :N].reshape(N, 1), indeg_p[1, :N].reshape(N, 1),
        outdeg_p[0, :N].reshape(N, 1), outdeg_p[1, :N].reshape(N, 1))

    agg_p = _sc_agg(row, col, Y)

    S, U, colsum, StS = _tc_b(
        agg_p[0, :N], agg_p[1, :N], xw, dinv, dinvl,
        b1.reshape(1, DG), fc1_W, fc1_b.reshape(1, D1),
        fc2_W, fc2_b.reshape(1, D2))

    t_p = _sc_lap(row, col, U)

    (UtT,) = _tc_c(U, t_p[0, :N], t_p[1, :N])

    new_adj = StS - UtT
    row_norm = jnp.sum(jnp.abs(new_adj), axis=1, keepdims=True)
    nrm = new_adj / jnp.maximum(row_norm, 1e-12)
    d = jnp.diag(nrm)
    pos_penalty = jnp.mean((d - jnp.eye(D2, dtype=jnp.float32)) ** 2)
    graph_embedding = colsum / jnp.float32(D2)
    return (graph_embedding, pos_penalty)


# speedup vs baseline: 31.7961x; 2.0103x over previous
import functools

import jax
import jax.numpy as jnp
from jax import lax
from jax.experimental import pallas as pl
from jax.experimental.pallas import tpu as pltpu
from jax.experimental.pallas import tpu_sc as plsc

N = 10000
E = 320000
DF = 128
DG = 128
D1 = 64
D2 = 16

NC = 2
NS = 16
NW = NC * NS
EPW = E // NW
NPAD = 10240
SPW = NPAD // NS

CHUNK = 80
NCHUNK = EPW // CHUNK


def _mesh():
    return plsc.VectorSubcoreMesh(core_axis_name="c", subcore_axis_name="s")


def _zero_ref(ref, nrows, ncols16):
    def body(r, _):
        for c in range(ncols16):
            ref[r, pl.ds(c * 16, 16)] = jnp.zeros((16,), jnp.float32)
        return 0
    lax.fori_loop(0, nrows, body, 0, unroll=4)


@functools.cache
def _make_sc_degrees():
    @functools.partial(
        pl.kernel,
        mesh=_mesh(),
        compiler_params=pltpu.CompilerParams(use_tc_tiling_on_sc=False),
        out_type=(
            jax.ShapeDtypeStruct((NC, NPAD), jnp.float32),
            jax.ShapeDtypeStruct((NC, NPAD), jnp.float32),
        ),
        scratch_types=(
            pltpu.VMEM_SHARED((NPAD,), jnp.float32),
            pltpu.VMEM_SHARED((NPAD,), jnp.float32),
            pltpu.VMEM((EPW,), jnp.int32),
            pltpu.VMEM((EPW,), jnp.int32),
            pltpu.VMEM((EPW,), jnp.float32),
            pltpu.VMEM((SPW,), jnp.float32),
        ),
    )
    def sc_degrees(row_hbm, col_hbm, indeg_hbm, outdeg_hbm,
                   in_acc, out_acc, ridx, cidx, ones_v, zbuf):
        c = lax.axis_index("c")
        s = lax.axis_index("s")
        wid = c * NS + s
        base = wid * EPW
        pltpu.sync_copy(row_hbm.at[pl.ds(base, EPW)], ridx)
        pltpu.sync_copy(col_hbm.at[pl.ds(base, EPW)], cidx)

        def fill(j, _):
            ones_v[pl.ds(j * 16, 16)] = jnp.ones((16,), jnp.float32)
            return 0

        lax.fori_loop(0, EPW // 16, fill, 0, unroll=4)
        for j in range(SPW // 16):
            zbuf[pl.ds(j * 16, 16)] = jnp.zeros((16,), jnp.float32)
        pltpu.sync_copy(zbuf, in_acc.at[pl.ds(s * SPW, SPW)])
        pltpu.sync_copy(zbuf, out_acc.at[pl.ds(s * SPW, SPW)])
        plsc.subcore_barrier()

        pltpu.sync_copy(ones_v, out_acc.at[ridx], add=True)
        pltpu.sync_copy(ones_v, in_acc.at[cidx], add=True)
        plsc.subcore_barrier()
        pltpu.sync_copy(in_acc.at[pl.ds(s * SPW, SPW)],
                        indeg_hbm.at[c, pl.ds(s * SPW, SPW)])
        pltpu.sync_copy(out_acc.at[pl.ds(s * SPW, SPW)],
                        outdeg_hbm.at[c, pl.ds(s * SPW, SPW)])

    return sc_degrees


def _sc_degrees(row, col):
    return _make_sc_degrees()(row, col)


ZROWS = SPW // 5


def _edge_pipeline(g_hbm, acc, ridx, cidx, rows_a, rows_b,
                   gsa, gsb, ssa, ssb):
    pltpu.async_copy(g_hbm.at[ridx.at[0]], rows_a, gsa)
    pltpu.make_async_copy(g_hbm.at[ridx.at[0]], rows_a, gsa).wait()
    pltpu.async_copy(rows_a, acc.at[cidx.at[0]], ssa, add=True)
    pltpu.async_copy(g_hbm.at[ridx.at[1]], rows_b, gsb)

    def body(j, _):
        c0 = 1 + 2 * j
        pltpu.make_async_copy(g_hbm.at[ridx.at[c0]], rows_b, gsb).wait()
        pltpu.async_copy(rows_b, acc.at[cidx.at[c0]], ssb, add=True)
        pltpu.make_async_copy(rows_a, acc.at[cidx.at[c0 - 1]], ssa).wait()
        pltpu.async_copy(g_hbm.at[ridx.at[c0 + 1]], rows_a, gsa)
        c1 = c0 + 1
        pltpu.make_async_copy(g_hbm.at[ridx.at[c1]], rows_a, gsa).wait()
        pltpu.async_copy(rows_a, acc.at[cidx.at[c1]], ssa, add=True)
        pltpu.make_async_copy(rows_b, acc.at[cidx.at[c1 - 1]], ssb).wait()

        @pl.when(c1 + 1 < NCHUNK)
        def _():
            pltpu.async_copy(g_hbm.at[ridx.at[c1 + 1]], rows_b, gsb)

        return 0

    lax.fori_loop(0, (NCHUNK - 1) // 2, body, 0)
    pltpu.make_async_copy(rows_a, acc.at[cidx.at[NCHUNK - 1]], ssa).wait()


@functools.cache
def _make_sc_agg():
    @functools.partial(
        pl.kernel,
        mesh=_mesh(),
        compiler_params=pltpu.CompilerParams(use_tc_tiling_on_sc=False),
        out_type=jax.ShapeDtypeStruct((NC, NPAD, DG), jnp.float32),
        scratch_types=(
            pltpu.VMEM_SHARED((NPAD, DG), jnp.float32),
            pltpu.VMEM((NCHUNK, CHUNK), jnp.int32),
            pltpu.VMEM((NCHUNK, CHUNK), jnp.int32),
            pltpu.VMEM((CHUNK, DG), jnp.float32),
            pltpu.VMEM((CHUNK, DG), jnp.float32),
            pltpu.SemaphoreType.DMA,
            pltpu.SemaphoreType.DMA,
            pltpu.SemaphoreType.DMA,
            pltpu.SemaphoreType.DMA,
        ),
    )
    def sc_agg(row_hbm, col_hbm, y_hbm, out_hbm,
               acc, ridx, cidx, rows_a, rows_b, gsa, gsb, ssa, ssb):
        c = lax.axis_index("c")
        s = lax.axis_index("s")
        wid = c * NS + s
        pltpu.sync_copy(row_hbm.at[wid], ridx)
        pltpu.sync_copy(col_hbm.at[wid], cidx)
        _zero_ref(rows_a, CHUNK, DG // 16)
        for j in range(SPW // CHUNK):
            pltpu.sync_copy(
                rows_a, acc.at[pl.ds(s * SPW + j * CHUNK, CHUNK), :])
        plsc.subcore_barrier()

        _edge_pipeline(y_hbm, acc, ridx, cidx, rows_a, rows_b,
                       gsa, gsb, ssa, ssb)

        plsc.subcore_barrier()
        pltpu.sync_copy(acc.at[pl.ds(s * SPW, SPW), :],
                        out_hbm.at[c, pl.ds(s * SPW, SPW), :])

    return sc_agg


def _sc_agg(row, col, Y):
    return _make_sc_agg()(row.reshape(NW, NCHUNK, CHUNK),
                          col.reshape(NW, NCHUNK, CHUNK), Y)


@functools.cache
def _make_sc_lap():
    @functools.partial(
        pl.kernel,
        mesh=_mesh(),
        compiler_params=pltpu.CompilerParams(use_tc_tiling_on_sc=False),
        out_type=jax.ShapeDtypeStruct((NC, NPAD, D2), jnp.float32),
        scratch_types=(
            pltpu.VMEM_SHARED((NPAD, D2), jnp.float32),
            pltpu.VMEM((NCHUNK, CHUNK), jnp.int32),
            pltpu.VMEM((NCHUNK, CHUNK), jnp.int32),
            pltpu.VMEM((CHUNK, D2), jnp.float32),
            pltpu.VMEM((CHUNK, D2), jnp.float32),
            pltpu.SemaphoreType.DMA,
            pltpu.SemaphoreType.DMA,
            pltpu.SemaphoreType.DMA,
            pltpu.SemaphoreType.DMA,
        ),
    )
    def sc_lap(row_hbm, col_hbm, u_hbm, out_hbm,
               acc, ridx, cidx, rows_a, rows_b, gsa, gsb, ssa, ssb):
        c = lax.axis_index("c")
        s = lax.axis_index("s")
        wid = c * NS + s
        pltpu.sync_copy(col_hbm.at[wid], ridx)
        pltpu.sync_copy(row_hbm.at[wid], cidx)
        _zero_ref(rows_a, CHUNK, D2 // 16)
        for j in range(SPW // CHUNK):
            pltpu.sync_copy(
                rows_a, acc.at[pl.ds(s * SPW + j * CHUNK, CHUNK), :])
        plsc.subcore_barrier()

        _edge_pipeline(u_hbm, acc, ridx, cidx, rows_a, rows_b,
                       gsa, gsb, ssa, ssb)

        plsc.subcore_barrier()
        pltpu.sync_copy(acc.at[pl.ds(s * SPW, SPW), :],
                        out_hbm.at[c, pl.ds(s * SPW, SPW), :])

    return sc_lap


def _sc_lap(row, col, U):
    return _make_sc_lap()(row.reshape(NW, NCHUNK, CHUNK),
                          col.reshape(NW, NCHUNK, CHUNK), U)


RB = 1000


def _tca_body(feat, w1, indeg0, indeg1, outdeg0, outdeg1,
              xw_o, y_o, dinv_o, dinvl_o):
    xw = jnp.dot(feat[...], w1[...], preferred_element_type=jnp.float32)
    indeg = indeg0[...] + indeg1[...]
    outdeg = outdeg0[...] + outdeg1[...]
    dinv = lax.rsqrt(indeg + 1.0)
    dinvl = jnp.where(outdeg > 0, lax.rsqrt(jnp.maximum(outdeg, 1.0)), 0.0)
    xw_o[...] = xw
    y_o[...] = dinv * xw
    dinv_o[...] = dinv
    dinvl_o[...] = dinvl


def _tc_a(features, W1, indeg0, indeg1, outdeg0, outdeg1):
    g = N // RB
    deg_spec = pl.BlockSpec((RB, 1), lambda i: (i, 0))
    return pl.pallas_call(
        _tca_body,
        grid=(g,),
        in_specs=[
            pl.BlockSpec((RB, DF), lambda i: (i, 0)),
            pl.BlockSpec((DF, DG), lambda i: (0, 0)),
            deg_spec, deg_spec, deg_spec, deg_spec,
        ],
        out_specs=[
            pl.BlockSpec((RB, DG), lambda i: (i, 0)),
            pl.BlockSpec((RB, DG), lambda i: (i, 0)),
            deg_spec, deg_spec,
        ],
        out_shape=[
            jax.ShapeDtypeStruct((N, DG), jnp.float32),
            jax.ShapeDtypeStruct((N, DG), jnp.float32),
            jax.ShapeDtypeStruct((N, 1), jnp.float32),
            jax.ShapeDtypeStruct((N, 1), jnp.float32),
        ],
    )(features, W1, indeg0, indeg1, outdeg0, outdeg1)


def _tcb_body(agg0, agg1, xw, dinv, dinvl, b1, w1, bb1, w2, bb2,
              s_o, u_o, cs_o, sts_o):
    di = dinv[...]
    agg = agg0[...] + agg1[...]
    nf2 = di * agg + (di * di) * xw[...] + b1[...]
    h1 = jnp.tanh(jnp.dot(nf2, w1[...], preferred_element_type=jnp.float32)
                  + bb1[...])
    lg = jnp.dot(h1, w2[...], preferred_element_type=jnp.float32) + bb2[...]
    m = jnp.max(lg, axis=1, keepdims=True)
    ex = jnp.exp(lg - m)
    S = ex / jnp.sum(ex, axis=1, keepdims=True)
    s_o[...] = S
    u_o[...] = dinvl[...] * S

    @pl.when(pl.program_id(0) == 0)
    def _():
        cs_o[...] = jnp.zeros_like(cs_o)
        sts_o[...] = jnp.zeros_like(sts_o)

    cs_o[...] += jnp.sum(nf2, axis=0, keepdims=True)
    sts_o[...] += lax.dot_general(S, S, (((0,), (0,)), ((), ())),
                                  preferred_element_type=jnp.float32)


def _tc_b(agg0, agg1, xw, dinv, dinvl, b1, fc1_W, fc1_b, fc2_W, fc2_b):
    g = N // RB
    deg_spec = pl.BlockSpec((RB, 1), lambda i: (i, 0))
    return pl.pallas_call(
        _tcb_body,
        grid=(g,),
        in_specs=[
            pl.BlockSpec((RB, DG), lambda i: (i, 0)),
            pl.BlockSpec((RB, DG), lambda i: (i, 0)),
            pl.BlockSpec((RB, DG), lambda i: (i, 0)),
            deg_spec, deg_spec,
            pl.BlockSpec((1, DG), lambda i: (0, 0)),
            pl.BlockSpec((DG, D1), lambda i: (0, 0)),
            pl.BlockSpec((1, D1), lambda i: (0, 0)),
            pl.BlockSpec((D1, D2), lambda i: (0, 0)),
            pl.BlockSpec((1, D2), lambda i: (0, 0)),
        ],
        out_specs=[
            pl.BlockSpec((RB, D2), lambda i: (i, 0)),
            pl.BlockSpec((RB, D2), lambda i: (i, 0)),
            pl.BlockSpec((1, DG), lambda i: (0, 0)),
            pl.BlockSpec((D2, D2), lambda i: (0, 0)),
        ],
        out_shape=[
            jax.ShapeDtypeStruct((N, D2), jnp.float32),
            jax.ShapeDtypeStruct((N, D2), jnp.float32),
            jax.ShapeDtypeStruct((1, DG), jnp.float32),
            jax.ShapeDtypeStruct((D2, D2), jnp.float32),
        ],
    )(agg0, agg1, xw, dinv, dinvl, b1, fc1_W, fc1_b, fc2_W, fc2_b)


def _tcc_body(u, t0, t1, utt_o):
    t = t0[...] + t1[...]

    @pl.when(pl.program_id(0) == 0)
    def _():
        utt_o[...] = jnp.zeros_like(utt_o)

    utt_o[...] += lax.dot_general(u[...], t, (((0,), (0,)), ((), ())),
                                  preferred_element_type=jnp.float32)


def _tc_c(U, t0, t1):
    g = N // RB
    return pl.pallas_call(
        _tcc_body,
        grid=(g,),
        in_specs=[
            pl.BlockSpec((RB, D2), lambda i: (i, 0)),
            pl.BlockSpec((RB, D2), lambda i: (i, 0)),
            pl.BlockSpec((RB, D2), lambda i: (i, 0)),
        ],
        out_specs=[pl.BlockSpec((D2, D2), lambda i: (0, 0))],
        out_shape=[jax.ShapeDtypeStruct((D2, D2), jnp.float32)],
    )(U, t0, t1)


def kernel(features, edges, W1, b1, fc1_W, fc1_b, fc2_W, fc2_b):
    row = edges[0]
    col = edges[1]

    indeg_p, outdeg_p = _sc_degrees(row, col)

    xw, Y, dinv, dinvl = _tc_a(
        features, W1,
        indeg_p[0, :N].reshape(N, 1), indeg_p[1, :N].reshape(N, 1),
        outdeg_p[0, :N].reshape(N, 1), outdeg_p[1, :N].reshape(N, 1))

    agg_p = _sc_agg(row, col, Y)

    S, U, colsum, StS = _tc_b(
        agg_p[0, :N], agg_p[1, :N], xw, dinv, dinvl,
        b1.reshape(1, DG), fc1_W, fc1_b.reshape(1, D1),
        fc2_W, fc2_b.reshape(1, D2))

    t_p = _sc_lap(row, col, U)

    (UtT,) = _tc_c(U, t_p[0, :N], t_p[1, :N])

    new_adj = StS - UtT
    row_norm = jnp.sum(jnp.abs(new_adj), axis=1, keepdims=True)
    nrm = new_adj / jnp.maximum(row_norm, 1e-12)
    d = jnp.diag(nrm)
    pos_penalty = jnp.mean((d - jnp.eye(D2, dtype=jnp.float32)) ** 2)
    graph_embedding = colsum / jnp.float32(D2)
    return (graph_embedding, pos_penalty)


# speedup vs baseline: 38.3275x; 1.2054x over previous
import functools

import jax
import jax.numpy as jnp
from jax import lax
from jax.experimental import pallas as pl
from jax.experimental.pallas import tpu as pltpu
from jax.experimental.pallas import tpu_sc as plsc

N = 10000
E = 320000
DF = 128
DG = 128
D1 = 64
D2 = 16

NC = 2
NS = 16
NW = NC * NS
EPW = E // NW
NPAD = 10240
SPW = NPAD // NS

CHUNK = 80
NCHUNK = EPW // CHUNK


def _mesh():
    return plsc.VectorSubcoreMesh(core_axis_name="c", subcore_axis_name="s")


def _zero_ref(ref, nrows, ncols16):
    def body(r, _):
        for c in range(ncols16):
            ref[r, pl.ds(c * 16, 16)] = jnp.zeros((16,), jnp.float32)
        return 0
    lax.fori_loop(0, nrows, body, 0, unroll=4)


@functools.cache
def _make_sc_degrees():
    @functools.partial(
        pl.kernel,
        mesh=_mesh(),
        compiler_params=pltpu.CompilerParams(use_tc_tiling_on_sc=False),
        out_type=(
            jax.ShapeDtypeStruct((NC, NPAD), jnp.float32),
            jax.ShapeDtypeStruct((NC, NPAD), jnp.float32),
        ),
        scratch_types=(
            pltpu.VMEM_SHARED((NPAD,), jnp.float32),
            pltpu.VMEM_SHARED((NPAD,), jnp.float32),
            pltpu.VMEM((EPW,), jnp.int32),
            pltpu.VMEM((EPW,), jnp.int32),
            pltpu.VMEM((EPW,), jnp.float32),
            pltpu.VMEM((SPW,), jnp.float32),
        ),
    )
    def sc_degrees(row_hbm, col_hbm, indeg_hbm, outdeg_hbm,
                   in_acc, out_acc, ridx, cidx, ones_v, zbuf):
        c = lax.axis_index("c")
        s = lax.axis_index("s")
        wid = c * NS + s
        base = wid * EPW
        pltpu.sync_copy(row_hbm.at[pl.ds(base, EPW)], ridx)
        pltpu.sync_copy(col_hbm.at[pl.ds(base, EPW)], cidx)

        def fill(j, _):
            ones_v[pl.ds(j * 16, 16)] = jnp.ones((16,), jnp.float32)
            return 0

        lax.fori_loop(0, EPW // 16, fill, 0, unroll=4)
        for j in range(SPW // 16):
            zbuf[pl.ds(j * 16, 16)] = jnp.zeros((16,), jnp.float32)
        pltpu.sync_copy(zbuf, in_acc.at[pl.ds(s * SPW, SPW)])
        pltpu.sync_copy(zbuf, out_acc.at[pl.ds(s * SPW, SPW)])
        plsc.subcore_barrier()

        pltpu.sync_copy(ones_v, out_acc.at[ridx], add=True)
        pltpu.sync_copy(ones_v, in_acc.at[cidx], add=True)
        plsc.subcore_barrier()
        pltpu.sync_copy(in_acc.at[pl.ds(s * SPW, SPW)],
                        indeg_hbm.at[c, pl.ds(s * SPW, SPW)])
        pltpu.sync_copy(out_acc.at[pl.ds(s * SPW, SPW)],
                        outdeg_hbm.at[c, pl.ds(s * SPW, SPW)])

    return sc_degrees


def _sc_degrees(row, col):
    return _make_sc_degrees()(row, col)


ZROWS = SPW // 5


def _edge_pipeline(g_hbm, acc, ridx, cidx, rows_a, rows_b,
                   gsa, gsb, ssa, ssb, nchunk):
    pltpu.async_copy(g_hbm.at[ridx.at[0]], rows_a, gsa)
    pltpu.make_async_copy(g_hbm.at[ridx.at[0]], rows_a, gsa).wait()
    pltpu.async_copy(rows_a, acc.at[cidx.at[0]], ssa, add=True)
    pltpu.async_copy(g_hbm.at[ridx.at[1]], rows_b, gsb)

    def body(j, _):
        c0 = 1 + 2 * j
        pltpu.make_async_copy(g_hbm.at[ridx.at[c0]], rows_b, gsb).wait()
        pltpu.async_copy(rows_b, acc.at[cidx.at[c0]], ssb, add=True)
        pltpu.make_async_copy(rows_a, acc.at[cidx.at[c0 - 1]], ssa).wait()
        pltpu.async_copy(g_hbm.at[ridx.at[c0 + 1]], rows_a, gsa)
        c1 = c0 + 1
        pltpu.make_async_copy(g_hbm.at[ridx.at[c1]], rows_a, gsa).wait()
        pltpu.async_copy(rows_a, acc.at[cidx.at[c1]], ssa, add=True)
        pltpu.make_async_copy(rows_b, acc.at[cidx.at[c1 - 1]], ssb).wait()

        @pl.when(c1 + 1 < nchunk)
        def _():
            pltpu.async_copy(g_hbm.at[ridx.at[c1 + 1]], rows_b, gsb)

        return 0

    lax.fori_loop(0, (nchunk - 1) // 2, body, 0)
    pltpu.make_async_copy(rows_a, acc.at[cidx.at[nchunk - 1]], ssa).wait()


@functools.cache
def _make_sc_agg():
    @functools.partial(
        pl.kernel,
        mesh=_mesh(),
        compiler_params=pltpu.CompilerParams(use_tc_tiling_on_sc=False),
        out_type=jax.ShapeDtypeStruct((NC, NPAD, DG), jnp.float32),
        scratch_types=(
            pltpu.VMEM_SHARED((NPAD, DG), jnp.float32),
            pltpu.VMEM((NCHUNK, CHUNK), jnp.int32),
            pltpu.VMEM((NCHUNK, CHUNK), jnp.int32),
            pltpu.VMEM((CHUNK, DG), jnp.float32),
            pltpu.VMEM((CHUNK, DG), jnp.float32),
            pltpu.SemaphoreType.DMA,
            pltpu.SemaphoreType.DMA,
            pltpu.SemaphoreType.DMA,
            pltpu.SemaphoreType.DMA,
        ),
    )
    def sc_agg(row_hbm, col_hbm, y_hbm, out_hbm,
               acc, ridx, cidx, rows_a, rows_b, gsa, gsb, ssa, ssb):
        c = lax.axis_index("c")
        s = lax.axis_index("s")
        wid = c * NS + s
        pltpu.sync_copy(row_hbm.at[wid], ridx)
        pltpu.sync_copy(col_hbm.at[wid], cidx)
        _zero_ref(rows_a, CHUNK, DG // 16)
        for j in range(SPW // CHUNK):
            pltpu.sync_copy(
                rows_a, acc.at[pl.ds(s * SPW + j * CHUNK, CHUNK), :])
        plsc.subcore_barrier()

        _edge_pipeline(y_hbm, acc, ridx, cidx, rows_a, rows_b,
                       gsa, gsb, ssa, ssb, NCHUNK)

        plsc.subcore_barrier()
        pltpu.sync_copy(acc.at[pl.ds(s * SPW, SPW), :],
                        out_hbm.at[c, pl.ds(s * SPW, SPW), :])

    return sc_agg


def _sc_agg(row, col, Y):
    return _make_sc_agg()(row.reshape(NW, NCHUNK, CHUNK),
                          col.reshape(NW, NCHUNK, CHUNK), Y)


LCHUNK = 400
LNCHUNK = EPW // LCHUNK


@functools.cache
def _make_sc_lap():
    @functools.partial(
        pl.kernel,
        mesh=_mesh(),
        compiler_params=pltpu.CompilerParams(use_tc_tiling_on_sc=False),
        out_type=jax.ShapeDtypeStruct((NC, NPAD, D2), jnp.float32),
        scratch_types=(
            pltpu.VMEM_SHARED((NPAD, D2), jnp.float32),
            pltpu.VMEM((LNCHUNK, LCHUNK), jnp.int32),
            pltpu.VMEM((LNCHUNK, LCHUNK), jnp.int32),
            pltpu.VMEM((LCHUNK, D2), jnp.float32),
            pltpu.VMEM((LCHUNK, D2), jnp.float32),
            pltpu.SemaphoreType.DMA,
            pltpu.SemaphoreType.DMA,
            pltpu.SemaphoreType.DMA,
            pltpu.SemaphoreType.DMA,
        ),
    )
    def sc_lap(row_hbm, col_hbm, u_hbm, out_hbm,
               acc, ridx, cidx, rows_a, rows_b, gsa, gsb, ssa, ssb):
        c = lax.axis_index("c")
        s = lax.axis_index("s")
        wid = c * NS + s
        pltpu.sync_copy(col_hbm.at[wid], ridx)
        pltpu.sync_copy(row_hbm.at[wid], cidx)
        _zero_ref(rows_a, LCHUNK, D2 // 16)
        pltpu.sync_copy(rows_a,
                        acc.at[pl.ds(s * SPW, LCHUNK), :])
        pltpu.sync_copy(rows_a.at[pl.ds(0, SPW - LCHUNK), :],
                        acc.at[pl.ds(s * SPW + LCHUNK, SPW - LCHUNK), :])
        plsc.subcore_barrier()

        _edge_pipeline(u_hbm, acc, ridx, cidx, rows_a, rows_b,
                       gsa, gsb, ssa, ssb, LNCHUNK)

        plsc.subcore_barrier()
        pltpu.sync_copy(acc.at[pl.ds(s * SPW, SPW), :],
                        out_hbm.at[c, pl.ds(s * SPW, SPW), :])

    return sc_lap


def _sc_lap(row, col, U):
    return _make_sc_lap()(row.reshape(NW, LNCHUNK, LCHUNK),
                          col.reshape(NW, LNCHUNK, LCHUNK), U)


RB = 1000


def _tca_body(feat, w1, indeg_p, outdeg_p,
              xw_o, y_o, dinv_o, dinvl_o):
    xw = jnp.dot(feat[...], w1[...], preferred_element_type=jnp.float32)
    indeg = indeg_p[0] + indeg_p[1]
    outdeg = outdeg_p[0] + outdeg_p[1]
    dinv = lax.rsqrt(indeg + 1.0)
    dinvl = jnp.where(outdeg > 0, lax.rsqrt(jnp.maximum(outdeg, 1.0)), 0.0)
    xw_o[...] = xw
    y_o[...] = dinv * xw
    dinv_o[...] = dinv
    dinvl_o[...] = dinvl


def _tc_a(features, W1, indeg_p, outdeg_p):
    g = N // RB
    degp_spec = pl.BlockSpec((NC, RB, 1), lambda i: (0, i, 0))
    deg_spec = pl.BlockSpec((RB, 1), lambda i: (i, 0))
    return pl.pallas_call(
        _tca_body,
        grid=(g,),
        in_specs=[
            pl.BlockSpec((RB, DF), lambda i: (i, 0)),
            pl.BlockSpec((DF, DG), lambda i: (0, 0)),
            degp_spec, degp_spec,
        ],
        out_specs=[
            pl.BlockSpec((RB, DG), lambda i: (i, 0)),
            pl.BlockSpec((RB, DG), lambda i: (i, 0)),
            deg_spec, deg_spec,
        ],
        out_shape=[
            jax.ShapeDtypeStruct((N, DG), jnp.float32),
            jax.ShapeDtypeStruct((N, DG), jnp.float32),
            jax.ShapeDtypeStruct((N, 1), jnp.float32),
            jax.ShapeDtypeStruct((N, 1), jnp.float32),
        ],
    )(features, W1, indeg_p.reshape(NC, NPAD, 1),
      outdeg_p.reshape(NC, NPAD, 1))


def _tcb_body(agg_p, xw, dinv, dinvl, b1, w1, bb1, w2, bb2,
              u_o, cs_o, sts_o):
    di = dinv[...]
    agg = agg_p[0] + agg_p[1]
    nf2 = di * agg + (di * di) * xw[...] + b1[...]
    h1 = jnp.tanh(jnp.dot(nf2, w1[...], preferred_element_type=jnp.float32)
                  + bb1[...])
    lg = jnp.dot(h1, w2[...], preferred_element_type=jnp.float32) + bb2[...]
    m = jnp.max(lg, axis=1, keepdims=True)
    ex = jnp.exp(lg - m)
    S = ex / jnp.sum(ex, axis=1, keepdims=True)
    u_o[...] = dinvl[...] * S

    @pl.when(pl.program_id(0) == 0)
    def _():
        cs_o[...] = jnp.zeros_like(cs_o)
        sts_o[...] = jnp.zeros_like(sts_o)

    cs_o[...] += jnp.sum(nf2, axis=0, keepdims=True)
    sts_o[...] += lax.dot_general(S, S, (((0,), (0,)), ((), ())),
                                  preferred_element_type=jnp.float32)


def _tc_b(agg_p, xw, dinv, dinvl, b1, fc1_W, fc1_b, fc2_W, fc2_b):
    g = N // RB
    deg_spec = pl.BlockSpec((RB, 1), lambda i: (i, 0))
    return pl.pallas_call(
        _tcb_body,
        grid=(g,),
        in_specs=[
            pl.BlockSpec((NC, RB, DG), lambda i: (0, i, 0)),
            pl.BlockSpec((RB, DG), lambda i: (i, 0)),
            deg_spec, deg_spec,
            pl.BlockSpec((1, DG), lambda i: (0, 0)),
            pl.BlockSpec((DG, D1), lambda i: (0, 0)),
            pl.BlockSpec((1, D1), lambda i: (0, 0)),
            pl.BlockSpec((D1, D2), lambda i: (0, 0)),
            pl.BlockSpec((1, D2), lambda i: (0, 0)),
        ],
        out_specs=[
            pl.BlockSpec((RB, D2), lambda i: (i, 0)),
            pl.BlockSpec((1, DG), lambda i: (0, 0)),
            pl.BlockSpec((D2, D2), lambda i: (0, 0)),
        ],
        out_shape=[
            jax.ShapeDtypeStruct((N, D2), jnp.float32),
            jax.ShapeDtypeStruct((1, DG), jnp.float32),
            jax.ShapeDtypeStruct((D2, D2), jnp.float32),
        ],
    )(agg_p, xw, dinv, dinvl, b1, fc1_W, fc1_b, fc2_W, fc2_b)


def _tcc_body(u, t_p, sts, colsum, ge_o, pp_o, utt_o):
    t = t_p[0] + t_p[1]

    @pl.when(pl.program_id(0) == 0)
    def _():
        utt_o[...] = jnp.zeros_like(utt_o)

    utt_o[...] += lax.dot_general(u[...], t, (((0,), (0,)), ((), ())),
                                  preferred_element_type=jnp.float32)

    @pl.when(pl.program_id(0) == N // RB - 1)
    def _():
        new_adj = sts[...] - utt_o[...]
        row_norm = jnp.sum(jnp.abs(new_adj), axis=1, keepdims=True)
        nrm = new_adj / jnp.maximum(row_norm, 1e-12)
        r_i = lax.broadcasted_iota(jnp.int32, (D2, D2), 0)
        c_i = lax.broadcasted_iota(jnp.int32, (D2, D2), 1)
        eye = jnp.where(r_i == c_i, 1.0, 0.0).astype(jnp.float32)
        d = jnp.sum(nrm * eye, axis=0, keepdims=True)
        pp = jnp.mean((d - eye) ** 2)
        pp_o[...] = jnp.reshape(pp, (1, 1))
        ge_o[...] = colsum[...] * jnp.float32(1.0 / D2)


def _tc_c(U, t_p, StS, colsum):
    g = N // RB
    return pl.pallas_call(
        _tcc_body,
        grid=(g,),
        in_specs=[
            pl.BlockSpec((RB, D2), lambda i: (i, 0)),
            pl.BlockSpec((NC, RB, D2), lambda i: (0, i, 0)),
            pl.BlockSpec((D2, D2), lambda i: (0, 0)),
            pl.BlockSpec((1, DG), lambda i: (0, 0)),
        ],
        out_specs=[
            pl.BlockSpec((1, DG), lambda i: (0, 0)),
            pl.BlockSpec((1, 1), lambda i: (0, 0)),
            pl.BlockSpec((D2, D2), lambda i: (0, 0)),
        ],
        out_shape=[
            jax.ShapeDtypeStruct((1, DG), jnp.float32),
            jax.ShapeDtypeStruct((1, 1), jnp.float32),
            jax.ShapeDtypeStruct((D2, D2), jnp.float32),
        ],
    )(U, t_p, StS, colsum)


def kernel(features, edges, W1, b1, fc1_W, fc1_b, fc2_W, fc2_b):
    row = edges[0]
    col = edges[1]

    indeg_p, outdeg_p = _sc_degrees(row, col)

    xw, Y, dinv, dinvl = _tc_a(features, W1, indeg_p, outdeg_p)

    agg_p = _sc_agg(row, col, Y)

    U, colsum, StS = _tc_b(
        agg_p, xw, dinv, dinvl,
        b1.reshape(1, DG), fc1_W, fc1_b.reshape(1, D1),
        fc2_W, fc2_b.reshape(1, D2))

    t_p = _sc_lap(row, col, U)

    graph_embedding, pp, _ = _tc_c(U, t_p, StS, colsum)
    return (graph_embedding, pp.reshape(()))


# speedup vs baseline: 38.4590x; 1.0034x over previous
import functools

import jax
import jax.numpy as jnp
from jax import lax
from jax.experimental import pallas as pl
from jax.experimental.pallas import tpu as pltpu
from jax.experimental.pallas import tpu_sc as plsc

N = 10000
E = 320000
DF = 128
DG = 128
D1 = 64
D2 = 16

NC = 2
NS = 16
NW = NC * NS
EPW = E // NW
NPAD = 10240
SPW = NPAD // NS

CHUNK = 80
NCHUNK = EPW // CHUNK


def _mesh():
    return plsc.VectorSubcoreMesh(core_axis_name="c", subcore_axis_name="s")


def _zero_ref(ref, nrows, ncols16):
    def body(r, _):
        for c in range(ncols16):
            ref[r, pl.ds(c * 16, 16)] = jnp.zeros((16,), jnp.float32)
        return 0
    lax.fori_loop(0, nrows, body, 0, unroll=4)


@functools.cache
def _make_sc_degrees():
    @functools.partial(
        pl.kernel,
        mesh=_mesh(),
        compiler_params=pltpu.CompilerParams(use_tc_tiling_on_sc=False),
        out_type=(
            jax.ShapeDtypeStruct((NC, NPAD), jnp.float32),
            jax.ShapeDtypeStruct((NC, NPAD), jnp.float32),
        ),
        scratch_types=(
            pltpu.VMEM_SHARED((NPAD,), jnp.float32),
            pltpu.VMEM_SHARED((NPAD,), jnp.float32),
            pltpu.VMEM((EPW,), jnp.int32),
            pltpu.VMEM((EPW,), jnp.int32),
            pltpu.VMEM((EPW,), jnp.float32),
            pltpu.VMEM((SPW,), jnp.float32),
        ),
    )
    def sc_degrees(row_hbm, col_hbm, indeg_hbm, outdeg_hbm,
                   in_acc, out_acc, ridx, cidx, ones_v, zbuf):
        c = lax.axis_index("c")
        s = lax.axis_index("s")
        wid = c * NS + s
        pltpu.sync_copy(row_hbm.at[wid], ridx)
        pltpu.sync_copy(col_hbm.at[wid], cidx)

        def fill(j, _):
            ones_v[pl.ds(j * 16, 16)] = jnp.ones((16,), jnp.float32)
            return 0

        lax.fori_loop(0, EPW // 16, fill, 0, unroll=4)
        for j in range(SPW // 16):
            zbuf[pl.ds(j * 16, 16)] = jnp.zeros((16,), jnp.float32)
        pltpu.sync_copy(zbuf, in_acc.at[pl.ds(s * SPW, SPW)])
        pltpu.sync_copy(zbuf, out_acc.at[pl.ds(s * SPW, SPW)])
        plsc.subcore_barrier()

        pltpu.sync_copy(ones_v, out_acc.at[ridx], add=True)
        pltpu.sync_copy(ones_v, in_acc.at[cidx], add=True)
        plsc.subcore_barrier()
        pltpu.sync_copy(in_acc.at[pl.ds(s * SPW, SPW)],
                        indeg_hbm.at[c, pl.ds(s * SPW, SPW)])
        pltpu.sync_copy(out_acc.at[pl.ds(s * SPW, SPW)],
                        outdeg_hbm.at[c, pl.ds(s * SPW, SPW)])

    return sc_degrees


def _sc_degrees(row2, col2):
    return _make_sc_degrees()(row2, col2)


ZROWS = SPW // 5


def _edge_pipeline(g_hbm, acc, ridx, cidx, rows_a, rows_b,
                   gsa, gsb, ssa, ssb, chunk, nchunk):
    def ri(c):
        return ridx.at[pl.ds(c * chunk, chunk)]

    def ci(c):
        return cidx.at[pl.ds(c * chunk, chunk)]

    pltpu.async_copy(g_hbm.at[ri(0)], rows_a, gsa)
    pltpu.make_async_copy(g_hbm.at[ri(0)], rows_a, gsa).wait()
    pltpu.async_copy(rows_a, acc.at[ci(0)], ssa, add=True)
    pltpu.async_copy(g_hbm.at[ri(1)], rows_b, gsb)

    def body(j, _):
        c0 = 1 + 2 * j
        pltpu.make_async_copy(g_hbm.at[ri(c0)], rows_b, gsb).wait()
        pltpu.async_copy(rows_b, acc.at[ci(c0)], ssb, add=True)
        pltpu.make_async_copy(rows_a, acc.at[ci(c0 - 1)], ssa).wait()
        pltpu.async_copy(g_hbm.at[ri(c0 + 1)], rows_a, gsa)
        c1 = c0 + 1
        pltpu.make_async_copy(g_hbm.at[ri(c1)], rows_a, gsa).wait()
        pltpu.async_copy(rows_a, acc.at[ci(c1)], ssa, add=True)
        pltpu.make_async_copy(rows_b, acc.at[ci(c1 - 1)], ssb).wait()

        @pl.when(c1 + 1 < nchunk)
        def _():
            pltpu.async_copy(g_hbm.at[ri(c1 + 1)], rows_b, gsb)

        return 0

    lax.fori_loop(0, (nchunk - 1) // 2, body, 0)
    pltpu.make_async_copy(rows_a, acc.at[ci(nchunk - 1)], ssa).wait()


@functools.cache
def _make_sc_agg():
    @functools.partial(
        pl.kernel,
        mesh=_mesh(),
        compiler_params=pltpu.CompilerParams(use_tc_tiling_on_sc=False),
        out_type=jax.ShapeDtypeStruct((NC, NPAD, DG), jnp.float32),
        scratch_types=(
            pltpu.VMEM_SHARED((NPAD, DG), jnp.float32),
            pltpu.VMEM((EPW,), jnp.int32),
            pltpu.VMEM((EPW,), jnp.int32),
            pltpu.VMEM((CHUNK, DG), jnp.float32),
            pltpu.VMEM((CHUNK, DG), jnp.float32),
            pltpu.SemaphoreType.DMA,
            pltpu.SemaphoreType.DMA,
            pltpu.SemaphoreType.DMA,
            pltpu.SemaphoreType.DMA,
        ),
    )
    def sc_agg(row_hbm, col_hbm, y_hbm, out_hbm,
               acc, ridx, cidx, rows_a, rows_b, gsa, gsb, ssa, ssb):
        c = lax.axis_index("c")
        s = lax.axis_index("s")
        wid = c * NS + s
        pltpu.sync_copy(row_hbm.at[wid], ridx)
        pltpu.sync_copy(col_hbm.at[wid], cidx)
        _zero_ref(rows_a, CHUNK, DG // 16)
        for j in range(SPW // CHUNK):
            pltpu.sync_copy(
                rows_a, acc.at[pl.ds(s * SPW + j * CHUNK, CHUNK), :])
        plsc.subcore_barrier()

        _edge_pipeline(y_hbm, acc, ridx, cidx, rows_a, rows_b,
                       gsa, gsb, ssa, ssb, CHUNK, NCHUNK)

        plsc.subcore_barrier()
        pltpu.sync_copy(acc.at[pl.ds(s * SPW, SPW), :],
                        out_hbm.at[c, pl.ds(s * SPW, SPW), :])

    return sc_agg


def _sc_agg(row2, col2, Y):
    return _make_sc_agg()(row2, col2, Y)


LCHUNK = 400
LNCHUNK = EPW // LCHUNK


@functools.cache
def _make_sc_lap():
    @functools.partial(
        pl.kernel,
        mesh=_mesh(),
        compiler_params=pltpu.CompilerParams(use_tc_tiling_on_sc=False),
        out_type=jax.ShapeDtypeStruct((NC, NPAD, D2), jnp.float32),
        scratch_types=(
            pltpu.VMEM_SHARED((NPAD, D2), jnp.float32),
            pltpu.VMEM((EPW,), jnp.int32),
            pltpu.VMEM((EPW,), jnp.int32),
            pltpu.VMEM((LCHUNK, D2), jnp.float32),
            pltpu.VMEM((LCHUNK, D2), jnp.float32),
            pltpu.SemaphoreType.DMA,
            pltpu.SemaphoreType.DMA,
            pltpu.SemaphoreType.DMA,
            pltpu.SemaphoreType.DMA,
        ),
    )
    def sc_lap(row_hbm, col_hbm, u_hbm, out_hbm,
               acc, ridx, cidx, rows_a, rows_b, gsa, gsb, ssa, ssb):
        c = lax.axis_index("c")
        s = lax.axis_index("s")
        wid = c * NS + s
        pltpu.sync_copy(col_hbm.at[wid], ridx)
        pltpu.sync_copy(row_hbm.at[wid], cidx)
        _zero_ref(rows_a, LCHUNK, D2 // 16)
        pltpu.sync_copy(rows_a,
                        acc.at[pl.ds(s * SPW, LCHUNK), :])
        pltpu.sync_copy(rows_a.at[pl.ds(0, SPW - LCHUNK), :],
                        acc.at[pl.ds(s * SPW + LCHUNK, SPW - LCHUNK), :])
        plsc.subcore_barrier()

        _edge_pipeline(u_hbm, acc, ridx, cidx, rows_a, rows_b,
                       gsa, gsb, ssa, ssb, LCHUNK, LNCHUNK)

        plsc.subcore_barrier()
        pltpu.sync_copy(acc.at[pl.ds(s * SPW, SPW), :],
                        out_hbm.at[c, pl.ds(s * SPW, SPW), :])

    return sc_lap


def _sc_lap(row2, col2, U):
    return _make_sc_lap()(row2, col2, U)


RB = 1000
RBA = 2000


def _tca_body(feat, w1, indeg_p, outdeg_p,
              xw_o, y_o, dinv_o, dinvl_o):
    xw = jnp.dot(feat[...], w1[...], preferred_element_type=jnp.float32)
    indeg = indeg_p[0] + indeg_p[1]
    outdeg = outdeg_p[0] + outdeg_p[1]
    dinv = lax.rsqrt(indeg + 1.0)
    dinvl = jnp.where(outdeg > 0, lax.rsqrt(jnp.maximum(outdeg, 1.0)), 0.0)
    xw_o[...] = xw
    y_o[...] = dinv * xw
    dinv_o[...] = dinv
    dinvl_o[...] = dinvl


def _tc_a(features, W1, indeg_p, outdeg_p):
    g = N // RBA
    degp_spec = pl.BlockSpec((NC, RBA, 1), lambda i: (0, i, 0))
    deg_spec = pl.BlockSpec((RBA, 1), lambda i: (i, 0))
    return pl.pallas_call(
        _tca_body,
        grid=(g,),
        in_specs=[
            pl.BlockSpec((RBA, DF), lambda i: (i, 0)),
            pl.BlockSpec((DF, DG), lambda i: (0, 0)),
            degp_spec, degp_spec,
        ],
        out_specs=[
            pl.BlockSpec((RBA, DG), lambda i: (i, 0)),
            pl.BlockSpec((RBA, DG), lambda i: (i, 0)),
            deg_spec, deg_spec,
        ],
        out_shape=[
            jax.ShapeDtypeStruct((N, DG), jnp.float32),
            jax.ShapeDtypeStruct((N, DG), jnp.float32),
            jax.ShapeDtypeStruct((N, 1), jnp.float32),
            jax.ShapeDtypeStruct((N, 1), jnp.float32),
        ],
    )(features, W1, indeg_p.reshape(NC, NPAD, 1),
      outdeg_p.reshape(NC, NPAD, 1))


def _tcb_body(agg_p, xw, dinv, dinvl, b1, w1, bb1, w2, bb2,
              u_o, cs_o, sts_o):
    di = dinv[...]
    agg = agg_p[0] + agg_p[1]
    nf2 = di * agg + (di * di) * xw[...] + b1[...]
    h1 = jnp.tanh(jnp.dot(nf2, w1[...], preferred_element_type=jnp.float32)
                  + bb1[...])
    lg = jnp.dot(h1, w2[...], preferred_element_type=jnp.float32) + bb2[...]
    m = jnp.max(lg, axis=1, keepdims=True)
    ex = jnp.exp(lg - m)
    S = ex / jnp.sum(ex, axis=1, keepdims=True)
    u_o[...] = dinvl[...] * S

    @pl.when(pl.program_id(0) == 0)
    def _():
        cs_o[...] = jnp.zeros_like(cs_o)
        sts_o[...] = jnp.zeros_like(sts_o)

    cs_o[...] += jnp.sum(nf2, axis=0, keepdims=True)
    sts_o[...] += lax.dot_general(S, S, (((0,), (0,)), ((), ())),
                                  preferred_element_type=jnp.float32)


def _tc_b(agg_p, xw, dinv, dinvl, b1, fc1_W, fc1_b, fc2_W, fc2_b):
    g = N // RB
    deg_spec = pl.BlockSpec((RB, 1), lambda i: (i, 0))
    return pl.pallas_call(
        _tcb_body,
        grid=(g,),
        in_specs=[
            pl.BlockSpec((NC, RB, DG), lambda i: (0, i, 0)),
            pl.BlockSpec((RB, DG), lambda i: (i, 0)),
            deg_spec, deg_spec,
            pl.BlockSpec((1, DG), lambda i: (0, 0)),
            pl.BlockSpec((DG, D1), lambda i: (0, 0)),
            pl.BlockSpec((1, D1), lambda i: (0, 0)),
            pl.BlockSpec((D1, D2), lambda i: (0, 0)),
            pl.BlockSpec((1, D2), lambda i: (0, 0)),
        ],
        out_specs=[
            pl.BlockSpec((RB, D2), lambda i: (i, 0)),
            pl.BlockSpec((1, DG), lambda i: (0, 0)),
            pl.BlockSpec((D2, D2), lambda i: (0, 0)),
        ],
        out_shape=[
            jax.ShapeDtypeStruct((N, D2), jnp.float32),
            jax.ShapeDtypeStruct((1, DG), jnp.float32),
            jax.ShapeDtypeStruct((D2, D2), jnp.float32),
        ],
    )(agg_p, xw, dinv, dinvl, b1, fc1_W, fc1_b, fc2_W, fc2_b)


def _tcc_body(u, t_p, sts, colsum, ge_o, pp_o, utt_o):
    t = t_p[0] + t_p[1]

    @pl.when(pl.program_id(0) == 0)
    def _():
        utt_o[...] = jnp.zeros_like(utt_o)

    utt_o[...] += lax.dot_general(u[...], t, (((0,), (0,)), ((), ())),
                                  preferred_element_type=jnp.float32)

    @pl.when(pl.program_id(0) == N // RB - 1)
    def _():
        new_adj = sts[...] - utt_o[...]
        row_norm = jnp.sum(jnp.abs(new_adj), axis=1, keepdims=True)
        nrm = new_adj / jnp.maximum(row_norm, 1e-12)
        r_i = lax.broadcasted_iota(jnp.int32, (D2, D2), 0)
        c_i = lax.broadcasted_iota(jnp.int32, (D2, D2), 1)
        eye = jnp.where(r_i == c_i, 1.0, 0.0).astype(jnp.float32)
        d = jnp.sum(nrm * eye, axis=0, keepdims=True)
        pp = jnp.mean((d - eye) ** 2)
        pp_o[...] = jnp.reshape(pp, (1, 1))
        ge_o[...] = colsum[...] * jnp.float32(1.0 / D2)


def _tc_c(U, t_p, StS, colsum):
    g = N // RB
    return pl.pallas_call(
        _tcc_body,
        grid=(g,),
        in_specs=[
            pl.BlockSpec((RB, D2), lambda i: (i, 0)),
            pl.BlockSpec((NC, RB, D2), lambda i: (0, i, 0)),
            pl.BlockSpec((D2, D2), lambda i: (0, 0)),
            pl.BlockSpec((1, DG), lambda i: (0, 0)),
        ],
        out_specs=[
            pl.BlockSpec((1, DG), lambda i: (0, 0)),
            pl.BlockSpec((1, 1), lambda i: (0, 0)),
            pl.BlockSpec((D2, D2), lambda i: (0, 0)),
        ],
        out_shape=[
            jax.ShapeDtypeStruct((1, DG), jnp.float32),
            jax.ShapeDtypeStruct((1, 1), jnp.float32),
            jax.ShapeDtypeStruct((D2, D2), jnp.float32),
        ],
    )(U, t_p, StS, colsum)


def kernel(features, edges, W1, b1, fc1_W, fc1_b, fc2_W, fc2_b):
    row = edges[0].reshape(NW, EPW)
    col = edges[1].reshape(NW, EPW)

    indeg_p, outdeg_p = _sc_degrees(row, col)

    xw, Y, dinv, dinvl = _tc_a(features, W1, indeg_p, outdeg_p)

    agg_p = _sc_agg(row, col, Y)

    U, colsum, StS = _tc_b(
        agg_p, xw, dinv, dinvl,
        b1.reshape(1, DG), fc1_W, fc1_b.reshape(1, D1),
        fc2_W, fc2_b.reshape(1, D2))

    t_p = _sc_lap(row, col, U)

    graph_embedding, pp, _ = _tc_c(U, t_p, StS, colsum)
    return (graph_embedding, pp.reshape(()))


# speedup vs baseline: 39.8234x; 1.0355x over previous
import functools

import jax
import jax.numpy as jnp
from jax import lax
from jax.experimental import pallas as pl
from jax.experimental.pallas import tpu as pltpu
from jax.experimental.pallas import tpu_sc as plsc

N = 10000
E = 320000
DF = 128
DG = 128
D1 = 64
D2 = 16

NC = 2
NS = 16
NW = NC * NS
EPW = E // NW
NPAD = 10240
SPW = NPAD // NS

CHUNK = 80
NCHUNK = EPW // CHUNK


def _mesh():
    return plsc.VectorSubcoreMesh(core_axis_name="c", subcore_axis_name="s")


def _zero_ref(ref, nrows, ncols16):
    def body(r, _):
        for c in range(ncols16):
            ref[r, pl.ds(c * 16, 16)] = jnp.zeros((16,), jnp.float32)
        return 0
    lax.fori_loop(0, nrows, body, 0, unroll=4)


@functools.cache
def _make_sc_degrees():
    @functools.partial(
        pl.kernel,
        mesh=_mesh(),
        compiler_params=pltpu.CompilerParams(use_tc_tiling_on_sc=False),
        out_type=(
            jax.ShapeDtypeStruct((NC, NPAD), jnp.float32),
            jax.ShapeDtypeStruct((NC, NPAD), jnp.float32),
        ),
        scratch_types=(
            pltpu.VMEM_SHARED((NPAD,), jnp.float32),
            pltpu.VMEM_SHARED((NPAD,), jnp.float32),
            pltpu.VMEM((EPW,), jnp.int32),
            pltpu.VMEM((EPW,), jnp.int32),
            pltpu.VMEM((EPW,), jnp.float32),
            pltpu.VMEM((SPW,), jnp.float32),
        ),
    )
    def sc_degrees(e_hbm, indeg_hbm, outdeg_hbm,
                   in_acc, out_acc, ridx, cidx, ones_v, zbuf):
        c = lax.axis_index("c")
        s = lax.axis_index("s")
        wid = c * NS + s
        pltpu.sync_copy(e_hbm.at[0, wid], ridx)
        pltpu.sync_copy(e_hbm.at[1, wid], cidx)

        def fill(j, _):
            ones_v[pl.ds(j * 16, 16)] = jnp.ones((16,), jnp.float32)
            return 0

        lax.fori_loop(0, EPW // 16, fill, 0, unroll=4)
        for j in range(SPW // 16):
            zbuf[pl.ds(j * 16, 16)] = jnp.zeros((16,), jnp.float32)
        pltpu.sync_copy(zbuf, in_acc.at[pl.ds(s * SPW, SPW)])
        pltpu.sync_copy(zbuf, out_acc.at[pl.ds(s * SPW, SPW)])
        plsc.subcore_barrier()

        pltpu.sync_copy(ones_v, out_acc.at[ridx], add=True)
        pltpu.sync_copy(ones_v, in_acc.at[cidx], add=True)
        plsc.subcore_barrier()
        pltpu.sync_copy(in_acc.at[pl.ds(s * SPW, SPW)],
                        indeg_hbm.at[c, pl.ds(s * SPW, SPW)])
        pltpu.sync_copy(out_acc.at[pl.ds(s * SPW, SPW)],
                        outdeg_hbm.at[c, pl.ds(s * SPW, SPW)])

    return sc_degrees


def _sc_degrees(e2):
    return _make_sc_degrees()(e2)


ZROWS = SPW // 5


def _edge_pipeline(g_hbm, acc, ridx, cidx, rows_a, rows_b,
                   gsa, gsb, ssa, ssb, chunk, nchunk):
    def ri(c):
        return ridx.at[pl.ds(c * chunk, chunk)]

    def ci(c):
        return cidx.at[pl.ds(c * chunk, chunk)]

    pltpu.async_copy(g_hbm.at[ri(0)], rows_a, gsa)
    pltpu.make_async_copy(g_hbm.at[ri(0)], rows_a, gsa).wait()
    pltpu.async_copy(rows_a, acc.at[ci(0)], ssa, add=True)
    pltpu.async_copy(g_hbm.at[ri(1)], rows_b, gsb)

    def body(j, _):
        c0 = 1 + 2 * j
        pltpu.make_async_copy(g_hbm.at[ri(c0)], rows_b, gsb).wait()
        pltpu.async_copy(rows_b, acc.at[ci(c0)], ssb, add=True)
        pltpu.make_async_copy(rows_a, acc.at[ci(c0 - 1)], ssa).wait()
        pltpu.async_copy(g_hbm.at[ri(c0 + 1)], rows_a, gsa)
        c1 = c0 + 1
        pltpu.make_async_copy(g_hbm.at[ri(c1)], rows_a, gsa).wait()
        pltpu.async_copy(rows_a, acc.at[ci(c1)], ssa, add=True)
        pltpu.make_async_copy(rows_b, acc.at[ci(c1 - 1)], ssb).wait()

        @pl.when(c1 + 1 < nchunk)
        def _():
            pltpu.async_copy(g_hbm.at[ri(c1 + 1)], rows_b, gsb)

        return 0

    lax.fori_loop(0, (nchunk - 1) // 2, body, 0)
    pltpu.make_async_copy(rows_a, acc.at[ci(nchunk - 1)], ssa).wait()


@functools.cache
def _make_sc_agg():
    @functools.partial(
        pl.kernel,
        mesh=_mesh(),
        compiler_params=pltpu.CompilerParams(use_tc_tiling_on_sc=False),
        out_type=jax.ShapeDtypeStruct((NC, NPAD, DG), jnp.float32),
        scratch_types=(
            pltpu.VMEM_SHARED((NPAD, DG), jnp.float32),
            pltpu.VMEM((EPW,), jnp.int32),
            pltpu.VMEM((EPW,), jnp.int32),
            pltpu.VMEM((CHUNK, DG), jnp.float32),
            pltpu.VMEM((CHUNK, DG), jnp.float32),
            pltpu.SemaphoreType.DMA,
            pltpu.SemaphoreType.DMA,
            pltpu.SemaphoreType.DMA,
            pltpu.SemaphoreType.DMA,
        ),
    )
    def sc_agg(e_hbm, y_hbm, out_hbm,
               acc, ridx, cidx, rows_a, rows_b, gsa, gsb, ssa, ssb):
        c = lax.axis_index("c")
        s = lax.axis_index("s")
        wid = c * NS + s
        pltpu.sync_copy(e_hbm.at[0, wid], ridx)
        pltpu.sync_copy(e_hbm.at[1, wid], cidx)
        _zero_ref(rows_a, CHUNK, DG // 16)
        for j in range(SPW // CHUNK):
            pltpu.sync_copy(
                rows_a, acc.at[pl.ds(s * SPW + j * CHUNK, CHUNK), :])
        plsc.subcore_barrier()

        _edge_pipeline(y_hbm, acc, ridx, cidx, rows_a, rows_b,
                       gsa, gsb, ssa, ssb, CHUNK, NCHUNK)

        plsc.subcore_barrier()
        pltpu.sync_copy(acc.at[pl.ds(s * SPW, SPW), :],
                        out_hbm.at[c, pl.ds(s * SPW, SPW), :])

    return sc_agg


def _sc_agg(e2, Y):
    return _make_sc_agg()(e2, Y)


LCHUNK = 400
LNCHUNK = EPW // LCHUNK


@functools.cache
def _make_sc_lap():
    @functools.partial(
        pl.kernel,
        mesh=_mesh(),
        compiler_params=pltpu.CompilerParams(use_tc_tiling_on_sc=False),
        out_type=jax.ShapeDtypeStruct((NC, NPAD, D2), jnp.float32),
        scratch_types=(
            pltpu.VMEM_SHARED((NPAD, D2), jnp.float32),
            pltpu.VMEM((EPW,), jnp.int32),
            pltpu.VMEM((EPW,), jnp.int32),
            pltpu.VMEM((LCHUNK, D2), jnp.float32),
            pltpu.VMEM((LCHUNK, D2), jnp.float32),
            pltpu.SemaphoreType.DMA,
            pltpu.SemaphoreType.DMA,
            pltpu.SemaphoreType.DMA,
            pltpu.SemaphoreType.DMA,
        ),
    )
    def sc_lap(e_hbm, u_hbm, out_hbm,
               acc, ridx, cidx, rows_a, rows_b, gsa, gsb, ssa, ssb):
        c = lax.axis_index("c")
        s = lax.axis_index("s")
        wid = c * NS + s
        pltpu.sync_copy(e_hbm.at[1, wid], ridx)
        pltpu.sync_copy(e_hbm.at[0, wid], cidx)
        _zero_ref(rows_a, LCHUNK, D2 // 16)
        pltpu.sync_copy(rows_a,
                        acc.at[pl.ds(s * SPW, LCHUNK), :])
        pltpu.sync_copy(rows_a.at[pl.ds(0, SPW - LCHUNK), :],
                        acc.at[pl.ds(s * SPW + LCHUNK, SPW - LCHUNK), :])
        plsc.subcore_barrier()

        _edge_pipeline(u_hbm, acc, ridx, cidx, rows_a, rows_b,
                       gsa, gsb, ssa, ssb, LCHUNK, LNCHUNK)

        plsc.subcore_barrier()
        pltpu.sync_copy(acc.at[pl.ds(s * SPW, SPW), :],
                        out_hbm.at[c, pl.ds(s * SPW, SPW), :])

    return sc_lap


def _sc_lap(e2, U):
    return _make_sc_lap()(e2, U)


RB = 1000
RBA = 2000


def _tca_body(feat, w1, indeg_p, outdeg_p,
              xw_o, y_o, dinv_o, dinvl_o):
    xw = jnp.dot(feat[...], w1[...], preferred_element_type=jnp.float32)
    indeg = indeg_p[0] + indeg_p[1]
    outdeg = outdeg_p[0] + outdeg_p[1]
    dinv = lax.rsqrt(indeg + 1.0)
    dinvl = jnp.where(outdeg > 0, lax.rsqrt(jnp.maximum(outdeg, 1.0)), 0.0)
    xw_o[...] = xw
    y_o[...] = dinv * xw
    dinv_o[...] = dinv
    dinvl_o[...] = dinvl


def _tc_a(features, W1, indeg_p, outdeg_p):
    g = N // RBA
    degp_spec = pl.BlockSpec((NC, RBA, 1), lambda i: (0, i, 0))
    deg_spec = pl.BlockSpec((RBA, 1), lambda i: (i, 0))
    return pl.pallas_call(
        _tca_body,
        grid=(g,),
        in_specs=[
            pl.BlockSpec((RBA, DF), lambda i: (i, 0)),
            pl.BlockSpec((DF, DG), lambda i: (0, 0)),
            degp_spec, degp_spec,
        ],
        out_specs=[
            pl.BlockSpec((RBA, DG), lambda i: (i, 0)),
            pl.BlockSpec((RBA, DG), lambda i: (i, 0)),
            deg_spec, deg_spec,
        ],
        out_shape=[
            jax.ShapeDtypeStruct((N, DG), jnp.float32),
            jax.ShapeDtypeStruct((N, DG), jnp.float32),
            jax.ShapeDtypeStruct((N, 1), jnp.float32),
            jax.ShapeDtypeStruct((N, 1), jnp.float32),
        ],
    )(features, W1, indeg_p.reshape(NC, NPAD, 1),
      outdeg_p.reshape(NC, NPAD, 1))


def _tcb_body(agg_p, xw, dinv, dinvl, b1, w1, bb1, w2, bb2,
              u_o, cs_o, sts_o):
    di = dinv[...]
    agg = agg_p[0] + agg_p[1]
    nf2 = di * agg + (di * di) * xw[...] + b1[...]
    h1 = jnp.tanh(jnp.dot(nf2, w1[...], preferred_element_type=jnp.float32)
                  + bb1[...])
    lg = jnp.dot(h1, w2[...], preferred_element_type=jnp.float32) + bb2[...]
    m = jnp.max(lg, axis=1, keepdims=True)
    ex = jnp.exp(lg - m)
    S = ex / jnp.sum(ex, axis=1, keepdims=True)
    u_o[...] = dinvl[...] * S

    @pl.when(pl.program_id(0) == 0)
    def _():
        cs_o[...] = jnp.zeros_like(cs_o)
        sts_o[...] = jnp.zeros_like(sts_o)

    cs_o[...] += jnp.sum(nf2, axis=0, keepdims=True)
    sts_o[...] += lax.dot_general(S, S, (((0,), (0,)), ((), ())),
                                  preferred_element_type=jnp.float32)


def _tc_b(agg_p, xw, dinv, dinvl, b1, fc1_W, fc1_b, fc2_W, fc2_b):
    g = N // RB
    deg_spec = pl.BlockSpec((RB, 1), lambda i: (i, 0))
    return pl.pallas_call(
        _tcb_body,
        grid=(g,),
        in_specs=[
            pl.BlockSpec((NC, RB, DG), lambda i: (0, i, 0)),
            pl.BlockSpec((RB, DG), lambda i: (i, 0)),
            deg_spec, deg_spec,
            pl.BlockSpec((1, DG), lambda i: (0, 0)),
            pl.BlockSpec((DG, D1), lambda i: (0, 0)),
            pl.BlockSpec((1, D1), lambda i: (0, 0)),
            pl.BlockSpec((D1, D2), lambda i: (0, 0)),
            pl.BlockSpec((1, D2), lambda i: (0, 0)),
        ],
        out_specs=[
            pl.BlockSpec((RB, D2), lambda i: (i, 0)),
            pl.BlockSpec((1, DG), lambda i: (0, 0)),
            pl.BlockSpec((D2, D2), lambda i: (0, 0)),
        ],
        out_shape=[
            jax.ShapeDtypeStruct((N, D2), jnp.float32),
            jax.ShapeDtypeStruct((1, DG), jnp.float32),
            jax.ShapeDtypeStruct((D2, D2), jnp.float32),
        ],
    )(agg_p, xw, dinv, dinvl, b1, fc1_W, fc1_b, fc2_W, fc2_b)


def _tcc_body(u, t_p, sts, colsum, ge_o, pp_o, utt_o):
    t = t_p[0] + t_p[1]

    @pl.when(pl.program_id(0) == 0)
    def _():
        utt_o[...] = jnp.zeros_like(utt_o)

    utt_o[...] += lax.dot_general(u[...], t, (((0,), (0,)), ((), ())),
                                  preferred_element_type=jnp.float32)

    @pl.when(pl.program_id(0) == N // RB - 1)
    def _():
        new_adj = sts[...] - utt_o[...]
        row_norm = jnp.sum(jnp.abs(new_adj), axis=1, keepdims=True)
        nrm = new_adj / jnp.maximum(row_norm, 1e-12)
        r_i = lax.broadcasted_iota(jnp.int32, (D2, D2), 0)
        c_i = lax.broadcasted_iota(jnp.int32, (D2, D2), 1)
        eye = jnp.where(r_i == c_i, 1.0, 0.0).astype(jnp.float32)
        d = jnp.sum(nrm * eye, axis=0, keepdims=True)
        pp = jnp.mean((d - eye) ** 2)
        pp_o[...] = jnp.reshape(pp, (1, 1))
        ge_o[...] = colsum[...] * jnp.float32(1.0 / D2)


def _tc_c(U, t_p, StS, colsum):
    g = N // RB
    return pl.pallas_call(
        _tcc_body,
        grid=(g,),
        in_specs=[
            pl.BlockSpec((RB, D2), lambda i: (i, 0)),
            pl.BlockSpec((NC, RB, D2), lambda i: (0, i, 0)),
            pl.BlockSpec((D2, D2), lambda i: (0, 0)),
            pl.BlockSpec((1, DG), lambda i: (0, 0)),
        ],
        out_specs=[
            pl.BlockSpec((1, DG), lambda i: (0, 0)),
            pl.BlockSpec((1, 1), lambda i: (0, 0)),
            pl.BlockSpec((D2, D2), lambda i: (0, 0)),
        ],
        out_shape=[
            jax.ShapeDtypeStruct((1, DG), jnp.float32),
            jax.ShapeDtypeStruct((1, 1), jnp.float32),
            jax.ShapeDtypeStruct((D2, D2), jnp.float32),
        ],
    )(U, t_p, StS, colsum)


def kernel(features, edges, W1, b1, fc1_W, fc1_b, fc2_W, fc2_b):
    e2 = edges.reshape(2, NW, EPW)

    indeg_p, outdeg_p = _sc_degrees(e2)

    xw, Y, dinv, dinvl = _tc_a(features, W1, indeg_p, outdeg_p)

    agg_p = _sc_agg(e2, Y)

    U, colsum, StS = _tc_b(
        agg_p, xw, dinv, dinvl,
        b1.reshape(1, DG), fc1_W, fc1_b.reshape(1, D1),
        fc2_W, fc2_b.reshape(1, D2))

    t_p = _sc_lap(e2, U)

    graph_embedding, pp, _ = _tc_c(U, t_p, StS, colsum)
    return (graph_embedding, pp.reshape(()))


# speedup vs baseline: 41.6061x; 1.0448x over previous
import functools

import jax
import jax.numpy as jnp
from jax import lax
from jax.experimental import pallas as pl
from jax.experimental.pallas import tpu as pltpu
from jax.experimental.pallas import tpu_sc as plsc

N = 10000
E = 320000
DF = 128
DG = 128
D1 = 64
D2 = 16

NC = 2
NS = 16
NW = NC * NS
EPW = E // NW
NPAD = 10240
SPW = NPAD // NS

CHUNK = 80
NCHUNK = EPW // CHUNK


def _mesh():
    return plsc.VectorSubcoreMesh(core_axis_name="c", subcore_axis_name="s")


def _zero_ref(ref, nrows, ncols16):
    def body(r, _):
        for c in range(ncols16):
            ref[r, pl.ds(c * 16, 16)] = jnp.zeros((16,), jnp.float32)
        return 0
    lax.fori_loop(0, nrows, body, 0, unroll=4)


@functools.cache
def _make_sc_degrees():
    @functools.partial(
        pl.kernel,
        mesh=_mesh(),
        compiler_params=pltpu.CompilerParams(use_tc_tiling_on_sc=False),
        out_type=(
            jax.ShapeDtypeStruct((NC, NPAD), jnp.float32),
            jax.ShapeDtypeStruct((NC, NPAD), jnp.float32),
        ),
        scratch_types=(
            pltpu.VMEM_SHARED((NPAD,), jnp.float32),
            pltpu.VMEM_SHARED((NPAD,), jnp.float32),
            pltpu.VMEM((EPW,), jnp.int32),
            pltpu.VMEM((EPW,), jnp.int32),
            pltpu.VMEM((EPW,), jnp.float32),
            pltpu.VMEM((SPW,), jnp.float32),
        ),
    )
    def sc_degrees(e_hbm, indeg_hbm, outdeg_hbm,
                   in_acc, out_acc, ridx, cidx, ones_v, zbuf):
        c = lax.axis_index("c")
        s = lax.axis_index("s")
        wid = c * NS + s
        pltpu.sync_copy(e_hbm.at[0, wid], ridx)
        pltpu.sync_copy(e_hbm.at[1, wid], cidx)

        def fill(j, _):
            ones_v[pl.ds(j * 16, 16)] = jnp.ones((16,), jnp.float32)
            return 0

        lax.fori_loop(0, EPW // 16, fill, 0, unroll=4)
        for j in range(SPW // 16):
            zbuf[pl.ds(j * 16, 16)] = jnp.zeros((16,), jnp.float32)
        pltpu.sync_copy(zbuf, in_acc.at[pl.ds(s * SPW, SPW)])
        pltpu.sync_copy(zbuf, out_acc.at[pl.ds(s * SPW, SPW)])
        plsc.subcore_barrier()

        pltpu.sync_copy(ones_v, out_acc.at[ridx], add=True)
        pltpu.sync_copy(ones_v, in_acc.at[cidx], add=True)
        plsc.subcore_barrier()
        pltpu.sync_copy(in_acc.at[pl.ds(s * SPW, SPW)],
                        indeg_hbm.at[c, pl.ds(s * SPW, SPW)])
        pltpu.sync_copy(out_acc.at[pl.ds(s * SPW, SPW)],
                        outdeg_hbm.at[c, pl.ds(s * SPW, SPW)])

    return sc_degrees


def _sc_degrees(e2):
    return _make_sc_degrees()(e2)


ZROWS = SPW // 5


def _edge_pipeline(g_hbm, acc, ridx, cidx, rows_a, rows_b,
                   gsa, gsb, ssa, ssb, chunk, nchunk):
    def ri(c):
        return ridx.at[pl.ds(c * chunk, chunk)]

    def ci(c):
        return cidx.at[pl.ds(c * chunk, chunk)]

    pltpu.async_copy(g_hbm.at[ri(0)], rows_a, gsa)
    pltpu.make_async_copy(g_hbm.at[ri(0)], rows_a, gsa).wait()
    pltpu.async_copy(rows_a, acc.at[ci(0)], ssa, add=True)
    pltpu.async_copy(g_hbm.at[ri(1)], rows_b, gsb)

    def body(j, _):
        c0 = 1 + 2 * j
        pltpu.make_async_copy(g_hbm.at[ri(c0)], rows_b, gsb).wait()
        pltpu.async_copy(rows_b, acc.at[ci(c0)], ssb, add=True)
        pltpu.make_async_copy(rows_a, acc.at[ci(c0 - 1)], ssa).wait()
        pltpu.async_copy(g_hbm.at[ri(c0 + 1)], rows_a, gsa)
        c1 = c0 + 1
        pltpu.make_async_copy(g_hbm.at[ri(c1)], rows_a, gsa).wait()
        pltpu.async_copy(rows_a, acc.at[ci(c1)], ssa, add=True)
        pltpu.make_async_copy(rows_b, acc.at[ci(c1 - 1)], ssb).wait()

        @pl.when(c1 + 1 < nchunk)
        def _():
            pltpu.async_copy(g_hbm.at[ri(c1 + 1)], rows_b, gsb)

        return 0

    lax.fori_loop(0, (nchunk - 1) // 2, body, 0)
    pltpu.make_async_copy(rows_a, acc.at[ci(nchunk - 1)], ssa).wait()


@functools.cache
def _make_sc_agg():
    @functools.partial(
        pl.kernel,
        mesh=_mesh(),
        compiler_params=pltpu.CompilerParams(use_tc_tiling_on_sc=False),
        out_type=jax.ShapeDtypeStruct((NC, NPAD, DG), jnp.float32),
        scratch_types=(
            pltpu.VMEM_SHARED((NPAD, DG), jnp.float32),
            pltpu.VMEM((EPW,), jnp.int32),
            pltpu.VMEM((EPW,), jnp.int32),
            pltpu.VMEM((CHUNK, DG), jnp.float32),
            pltpu.VMEM((CHUNK, DG), jnp.float32),
            pltpu.SemaphoreType.DMA,
            pltpu.SemaphoreType.DMA,
            pltpu.SemaphoreType.DMA,
            pltpu.SemaphoreType.DMA,
        ),
    )
    def sc_agg(e_hbm, y_hbm, out_hbm,
               acc, ridx, cidx, rows_a, rows_b, gsa, gsb, ssa, ssb):
        c = lax.axis_index("c")
        s = lax.axis_index("s")
        wid = c * NS + s
        pltpu.sync_copy(e_hbm.at[0, wid], ridx)
        pltpu.sync_copy(e_hbm.at[1, wid], cidx)
        _zero_ref(rows_a, CHUNK, DG // 16)
        for j in range(SPW // CHUNK):
            pltpu.sync_copy(
                rows_a, acc.at[pl.ds(s * SPW + j * CHUNK, CHUNK), :])
        plsc.subcore_barrier()

        _edge_pipeline(y_hbm, acc, ridx, cidx, rows_a, rows_b,
                       gsa, gsb, ssa, ssb, CHUNK, NCHUNK)

        plsc.subcore_barrier()
        pltpu.sync_copy(acc.at[pl.ds(s * SPW, SPW), :],
                        out_hbm.at[c, pl.ds(s * SPW, SPW), :])

    return sc_agg


def _sc_agg(e2, Y):
    return _make_sc_agg()(e2, Y)


LCHUNK = 400
LNCHUNK = EPW // LCHUNK


@functools.cache
def _make_sc_lap():
    @functools.partial(
        pl.kernel,
        mesh=_mesh(),
        compiler_params=pltpu.CompilerParams(use_tc_tiling_on_sc=False),
        out_type=jax.ShapeDtypeStruct((NC, NPAD, D2), jnp.float32),
        scratch_types=(
            pltpu.VMEM_SHARED((NPAD, D2), jnp.float32),
            pltpu.VMEM((EPW,), jnp.int32),
            pltpu.VMEM((EPW,), jnp.int32),
            pltpu.VMEM((LCHUNK, D2), jnp.float32),
            pltpu.VMEM((LCHUNK, D2), jnp.float32),
            pltpu.SemaphoreType.DMA,
            pltpu.SemaphoreType.DMA,
            pltpu.SemaphoreType.DMA,
            pltpu.SemaphoreType.DMA,
        ),
    )
    def sc_lap(e_hbm, u_hbm, out_hbm,
               acc, ridx, cidx, rows_a, rows_b, gsa, gsb, ssa, ssb):
        c = lax.axis_index("c")
        s = lax.axis_index("s")
        wid = c * NS + s
        pltpu.sync_copy(e_hbm.at[1, wid], ridx)
        pltpu.sync_copy(e_hbm.at[0, wid], cidx)
        _zero_ref(rows_a, LCHUNK, D2 // 16)
        pltpu.sync_copy(rows_a,
                        acc.at[pl.ds(s * SPW, LCHUNK), :])
        pltpu.sync_copy(rows_a.at[pl.ds(0, SPW - LCHUNK), :],
                        acc.at[pl.ds(s * SPW + LCHUNK, SPW - LCHUNK), :])
        plsc.subcore_barrier()

        _edge_pipeline(u_hbm, acc, ridx, cidx, rows_a, rows_b,
                       gsa, gsb, ssa, ssb, LCHUNK, LNCHUNK)

        plsc.subcore_barrier()
        pltpu.sync_copy(acc.at[pl.ds(s * SPW, SPW), :],
                        out_hbm.at[c, pl.ds(s * SPW, SPW), :])

    return sc_lap


def _sc_lap(e2, U):
    return _make_sc_lap()(e2, U)


RB = 2000
RBA = 2000


def _tca_body(feat, w1, indeg_p, outdeg_p,
              y_o, dinv_o, dinvl_o):
    xw = jnp.dot(feat[...], w1[...], preferred_element_type=jnp.float32)
    indeg = indeg_p[0] + indeg_p[1]
    outdeg = outdeg_p[0] + outdeg_p[1]
    dinv = lax.rsqrt(indeg + 1.0)
    dinvl = jnp.where(outdeg > 0, lax.rsqrt(jnp.maximum(outdeg, 1.0)), 0.0)
    y_o[...] = dinv * xw
    dinv_o[...] = dinv
    dinvl_o[...] = dinvl


def _tc_a(features, W1, indeg_p, outdeg_p):
    g = N // RBA
    degp_spec = pl.BlockSpec((NC, RBA, 1), lambda i: (0, i, 0))
    deg_spec = pl.BlockSpec((RBA, 1), lambda i: (i, 0))
    return pl.pallas_call(
        _tca_body,
        grid=(g,),
        in_specs=[
            pl.BlockSpec((RBA, DF), lambda i: (i, 0)),
            pl.BlockSpec((DF, DG), lambda i: (0, 0)),
            degp_spec, degp_spec,
        ],
        out_specs=[
            pl.BlockSpec((RBA, DG), lambda i: (i, 0)),
            deg_spec, deg_spec,
        ],
        out_shape=[
            jax.ShapeDtypeStruct((N, DG), jnp.float32),
            jax.ShapeDtypeStruct((N, 1), jnp.float32),
            jax.ShapeDtypeStruct((N, 1), jnp.float32),
        ],
    )(features, W1, indeg_p.reshape(NC, NPAD, 1),
      outdeg_p.reshape(NC, NPAD, 1))


def _tcb_body(agg_p, y, dinv, dinvl, b1, w1, bb1, w2, bb2,
              u_o, cs_o, sts_o):
    di = dinv[...]
    agg = agg_p[0] + agg_p[1]
    nf2 = di * (agg + y[...]) + b1[...]
    h1 = jnp.tanh(jnp.dot(nf2, w1[...], preferred_element_type=jnp.float32)
                  + bb1[...])
    lg = jnp.dot(h1, w2[...], preferred_element_type=jnp.float32) + bb2[...]
    m = jnp.max(lg, axis=1, keepdims=True)
    ex = jnp.exp(lg - m)
    S = ex / jnp.sum(ex, axis=1, keepdims=True)
    u_o[...] = dinvl[...] * S

    @pl.when(pl.program_id(0) == 0)
    def _():
        cs_o[...] = jnp.zeros_like(cs_o)
        sts_o[...] = jnp.zeros_like(sts_o)

    cs_o[...] += jnp.sum(nf2, axis=0, keepdims=True)
    sts_o[...] += lax.dot_general(S, S, (((0,), (0,)), ((), ())),
                                  preferred_element_type=jnp.float32)


def _tc_b(agg_p, y, dinv, dinvl, b1, fc1_W, fc1_b, fc2_W, fc2_b):
    g = N // RB
    deg_spec = pl.BlockSpec((RB, 1), lambda i: (i, 0))
    return pl.pallas_call(
        _tcb_body,
        grid=(g,),
        in_specs=[
            pl.BlockSpec((NC, RB, DG), lambda i: (0, i, 0)),
            pl.BlockSpec((RB, DG), lambda i: (i, 0)),
            deg_spec, deg_spec,
            pl.BlockSpec((1, DG), lambda i: (0, 0)),
            pl.BlockSpec((DG, D1), lambda i: (0, 0)),
            pl.BlockSpec((1, D1), lambda i: (0, 0)),
            pl.BlockSpec((D1, D2), lambda i: (0, 0)),
            pl.BlockSpec((1, D2), lambda i: (0, 0)),
        ],
        out_specs=[
            pl.BlockSpec((RB, D2), lambda i: (i, 0)),
            pl.BlockSpec((1, DG), lambda i: (0, 0)),
            pl.BlockSpec((D2, D2), lambda i: (0, 0)),
        ],
        out_shape=[
            jax.ShapeDtypeStruct((N, D2), jnp.float32),
            jax.ShapeDtypeStruct((1, DG), jnp.float32),
            jax.ShapeDtypeStruct((D2, D2), jnp.float32),
        ],
    )(agg_p, y, dinv, dinvl, b1, fc1_W, fc1_b, fc2_W, fc2_b)


def _tcc_body(u, t_p, sts, colsum, ge_o, pp_o, utt_o):
    t = t_p[0] + t_p[1]

    @pl.when(pl.program_id(0) == 0)
    def _():
        utt_o[...] = jnp.zeros_like(utt_o)

    utt_o[...] += lax.dot_general(u[...], t, (((0,), (0,)), ((), ())),
                                  preferred_element_type=jnp.float32)

    @pl.when(pl.program_id(0) == N // RB - 1)
    def _():
        new_adj = sts[...] - utt_o[...]
        row_norm = jnp.sum(jnp.abs(new_adj), axis=1, keepdims=True)
        nrm = new_adj / jnp.maximum(row_norm, 1e-12)
        r_i = lax.broadcasted_iota(jnp.int32, (D2, D2), 0)
        c_i = lax.broadcasted_iota(jnp.int32, (D2, D2), 1)
        eye = jnp.where(r_i == c_i, 1.0, 0.0).astype(jnp.float32)
        d = jnp.sum(nrm * eye, axis=0, keepdims=True)
        pp = jnp.mean((d - eye) ** 2)
        pp_o[...] = jnp.reshape(pp, (1, 1))
        ge_o[...] = colsum[...] * jnp.float32(1.0 / D2)


def _tc_c(U, t_p, StS, colsum):
    g = N // RB
    return pl.pallas_call(
        _tcc_body,
        grid=(g,),
        in_specs=[
            pl.BlockSpec((RB, D2), lambda i: (i, 0)),
            pl.BlockSpec((NC, RB, D2), lambda i: (0, i, 0)),
            pl.BlockSpec((D2, D2), lambda i: (0, 0)),
            pl.BlockSpec((1, DG), lambda i: (0, 0)),
        ],
        out_specs=[
            pl.BlockSpec((1, DG), lambda i: (0, 0)),
            pl.BlockSpec((1, 1), lambda i: (0, 0)),
            pl.BlockSpec((D2, D2), lambda i: (0, 0)),
        ],
        out_shape=[
            jax.ShapeDtypeStruct((1, DG), jnp.float32),
            jax.ShapeDtypeStruct((1, 1), jnp.float32),
            jax.ShapeDtypeStruct((D2, D2), jnp.float32),
        ],
    )(U, t_p, StS, colsum)


def kernel(features, edges, W1, b1, fc1_W, fc1_b, fc2_W, fc2_b):
    e2 = edges.reshape(2, NW, EPW)

    indeg_p, outdeg_p = _sc_degrees(e2)

    Y, dinv, dinvl = _tc_a(features, W1, indeg_p, outdeg_p)

    agg_p = _sc_agg(e2, Y)

    U, colsum, StS = _tc_b(
        agg_p, Y, dinv, dinvl,
        b1.reshape(1, DG), fc1_W, fc1_b.reshape(1, D1),
        fc2_W, fc2_b.reshape(1, D2))

    t_p = _sc_lap(e2, U)

    graph_embedding, pp, _ = _tc_c(U, t_p, StS, colsum)
    return (graph_embedding, pp.reshape(()))


# speedup vs baseline: 46.6916x; 1.1222x over previous
import functools

import jax
import jax.numpy as jnp
from jax import lax
from jax.experimental import pallas as pl
from jax.experimental.pallas import tpu as pltpu
from jax.experimental.pallas import tpu_sc as plsc

N = 10000
E = 320000
DF = 128
DG = 128
D1 = 64
D2 = 16

NC = 2
NS = 16
NW = NC * NS
EPW = E // NW
NPAD = 10240
SPW = NPAD // NS

CHUNK = 80
NCHUNK = EPW // CHUNK


def _mesh():
    return plsc.VectorSubcoreMesh(core_axis_name="c", subcore_axis_name="s")


def _zero_ref(ref, nrows, ncols16):
    def body(r, _):
        for c in range(ncols16):
            ref[r, pl.ds(c * 16, 16)] = jnp.zeros((16,), jnp.float32)
        return 0
    lax.fori_loop(0, nrows, body, 0, unroll=4)


@functools.cache
def _make_sc_degrees():
    @functools.partial(
        pl.kernel,
        mesh=_mesh(),
        compiler_params=pltpu.CompilerParams(use_tc_tiling_on_sc=False),
        out_type=(
            jax.ShapeDtypeStruct((NC, NPAD), jnp.float32),
            jax.ShapeDtypeStruct((NC, NPAD), jnp.float32),
        ),
        scratch_types=(
            pltpu.VMEM_SHARED((NPAD,), jnp.float32),
            pltpu.VMEM_SHARED((NPAD,), jnp.float32),
            pltpu.VMEM((EPW,), jnp.int32),
            pltpu.VMEM((EPW,), jnp.int32),
            pltpu.VMEM((EPW,), jnp.float32),
            pltpu.VMEM((SPW,), jnp.float32),
        ),
    )
    def sc_degrees(e_hbm, indeg_hbm, outdeg_hbm,
                   in_acc, out_acc, ridx, cidx, ones_v, zbuf):
        c = lax.axis_index("c")
        s = lax.axis_index("s")
        wid = c * NS + s
        pltpu.sync_copy(e_hbm.at[0, wid], ridx)
        pltpu.sync_copy(e_hbm.at[1, wid], cidx)

        def fill(j, _):
            ones_v[pl.ds(j * 16, 16)] = jnp.ones((16,), jnp.float32)
            return 0

        lax.fori_loop(0, EPW // 16, fill, 0, unroll=4)
        for j in range(SPW // 16):
            zbuf[pl.ds(j * 16, 16)] = jnp.zeros((16,), jnp.float32)
        pltpu.sync_copy(zbuf, in_acc.at[pl.ds(s * SPW, SPW)])
        pltpu.sync_copy(zbuf, out_acc.at[pl.ds(s * SPW, SPW)])
        plsc.subcore_barrier()

        pltpu.sync_copy(ones_v, out_acc.at[ridx], add=True)
        pltpu.sync_copy(ones_v, in_acc.at[cidx], add=True)
        plsc.subcore_barrier()
        pltpu.sync_copy(in_acc.at[pl.ds(s * SPW, SPW)],
                        indeg_hbm.at[c, pl.ds(s * SPW, SPW)])
        pltpu.sync_copy(out_acc.at[pl.ds(s * SPW, SPW)],
                        outdeg_hbm.at[c, pl.ds(s * SPW, SPW)])

    return sc_degrees


def _sc_degrees(e2):
    return _make_sc_degrees()(e2)


ZROWS = SPW // 5


def _edge_pipeline(g_hbm, acc, ridx, cidx, rows_a, rows_b,
                   gsa, gsb, ssa, ssb, chunk, nchunk):
    def ri(c):
        return ridx.at[pl.ds(c * chunk, chunk)]

    def ci(c):
        return cidx.at[pl.ds(c * chunk, chunk)]

    pltpu.async_copy(g_hbm.at[ri(0)], rows_a, gsa)
    pltpu.make_async_copy(g_hbm.at[ri(0)], rows_a, gsa).wait()
    pltpu.async_copy(rows_a, acc.at[ci(0)], ssa, add=True)
    pltpu.async_copy(g_hbm.at[ri(1)], rows_b, gsb)

    def body(j, _):
        c0 = 1 + 2 * j
        pltpu.make_async_copy(g_hbm.at[ri(c0)], rows_b, gsb).wait()
        pltpu.async_copy(rows_b, acc.at[ci(c0)], ssb, add=True)
        pltpu.make_async_copy(rows_a, acc.at[ci(c0 - 1)], ssa).wait()
        pltpu.async_copy(g_hbm.at[ri(c0 + 1)], rows_a, gsa)
        c1 = c0 + 1
        pltpu.make_async_copy(g_hbm.at[ri(c1)], rows_a, gsa).wait()
        pltpu.async_copy(rows_a, acc.at[ci(c1)], ssa, add=True)
        pltpu.make_async_copy(rows_b, acc.at[ci(c1 - 1)], ssb).wait()

        @pl.when(c1 + 1 < nchunk)
        def _():
            pltpu.async_copy(g_hbm.at[ri(c1 + 1)], rows_b, gsb)

        return 0

    lax.fori_loop(0, (nchunk - 1) // 2, body, 0)
    pltpu.make_async_copy(rows_a, acc.at[ci(nchunk - 1)], ssa).wait()


@functools.cache
def _make_sc_agg():
    @functools.partial(
        pl.kernel,
        mesh=_mesh(),
        compiler_params=pltpu.CompilerParams(use_tc_tiling_on_sc=False),
        out_type=jax.ShapeDtypeStruct((NC, NPAD, DG), jnp.float32),
        scratch_types=(
            pltpu.VMEM_SHARED((NPAD, DG), jnp.float32),
            pltpu.VMEM((EPW,), jnp.int32),
            pltpu.VMEM((EPW,), jnp.int32),
            pltpu.VMEM((CHUNK, DG), jnp.float32),
            pltpu.VMEM((CHUNK, DG), jnp.float32),
            pltpu.SemaphoreType.DMA,
            pltpu.SemaphoreType.DMA,
            pltpu.SemaphoreType.DMA,
            pltpu.SemaphoreType.DMA,
        ),
    )
    def sc_agg(e_hbm, y_hbm, out_hbm,
               acc, ridx, cidx, rows_a, rows_b, gsa, gsb, ssa, ssb):
        c = lax.axis_index("c")
        s = lax.axis_index("s")
        wid = c * NS + s
        pltpu.sync_copy(e_hbm.at[0, wid], ridx)
        pltpu.sync_copy(e_hbm.at[1, wid], cidx)
        _zero_ref(rows_a, CHUNK, DG // 16)
        for j in range(SPW // CHUNK):
            pltpu.sync_copy(
                rows_a, acc.at[pl.ds(s * SPW + j * CHUNK, CHUNK), :])
        plsc.subcore_barrier()

        _edge_pipeline(y_hbm, acc, ridx, cidx, rows_a, rows_b,
                       gsa, gsb, ssa, ssb, CHUNK, NCHUNK)

        plsc.subcore_barrier()
        pltpu.sync_copy(acc.at[pl.ds(s * SPW, SPW), :],
                        out_hbm.at[c, pl.ds(s * SPW, SPW), :])

    return sc_agg


def _sc_agg(e2, Y):
    return _make_sc_agg()(e2, Y)


LCHUNK = 1000
LNCHUNK = EPW // LCHUNK


@functools.cache
def _make_sc_lap():
    @functools.partial(
        pl.kernel,
        mesh=_mesh(),
        compiler_params=pltpu.CompilerParams(use_tc_tiling_on_sc=False),
        out_type=jax.ShapeDtypeStruct((NC, NPAD, D2), jnp.float32),
        scratch_types=(
            pltpu.VMEM_SHARED((NPAD, D2), jnp.float32),
            pltpu.VMEM((EPW,), jnp.int32),
            pltpu.VMEM((EPW,), jnp.int32),
            pltpu.VMEM((LCHUNK, D2), jnp.float32),
            pltpu.VMEM((LCHUNK, D2), jnp.float32),
            pltpu.SemaphoreType.DMA,
            pltpu.SemaphoreType.DMA,
            pltpu.SemaphoreType.DMA,
            pltpu.SemaphoreType.DMA,
        ),
    )
    def sc_lap(e_hbm, u_hbm, out_hbm,
               acc, ridx, cidx, rows_a, rows_b, gsa, gsb, ssa, ssb):
        c = lax.axis_index("c")
        s = lax.axis_index("s")
        wid = c * NS + s
        pltpu.sync_copy(e_hbm.at[1, wid], ridx)
        pltpu.sync_copy(e_hbm.at[0, wid], cidx)
        _zero_ref(rows_a, SPW, D2 // 16)
        pltpu.sync_copy(rows_a.at[pl.ds(0, SPW), :],
                        acc.at[pl.ds(s * SPW, SPW), :])
        plsc.subcore_barrier()

        _edge_pipeline(u_hbm, acc, ridx, cidx, rows_a, rows_b,
                       gsa, gsb, ssa, ssb, LCHUNK, LNCHUNK)

        plsc.subcore_barrier()
        pltpu.sync_copy(acc.at[pl.ds(s * SPW, SPW), :],
                        out_hbm.at[c, pl.ds(s * SPW, SPW), :])

    return sc_lap


def _sc_lap(e2, U):
    return _make_sc_lap()(e2, U)


RB = 2000
RBA = 2000


def _tca_body(feat, w1, indeg_p, outdeg_p,
              y_o, dinv_o, dinvl_o):
    xw = jnp.dot(feat[...], w1[...], preferred_element_type=jnp.float32)
    indeg = indeg_p[0] + indeg_p[1]
    outdeg = outdeg_p[0] + outdeg_p[1]
    dinv = lax.rsqrt(indeg + 1.0)
    dinvl = jnp.where(outdeg > 0, lax.rsqrt(jnp.maximum(outdeg, 1.0)), 0.0)
    y_o[...] = dinv * xw
    dinv_o[...] = dinv
    dinvl_o[...] = dinvl


def _tc_a(features, W1, indeg_p, outdeg_p):
    g = N // RBA
    degp_spec = pl.BlockSpec((NC, RBA, 1), lambda i: (0, i, 0))
    deg_spec = pl.BlockSpec((RBA, 1), lambda i: (i, 0))
    return pl.pallas_call(
        _tca_body,
        grid=(g,),
        in_specs=[
            pl.BlockSpec((RBA, DF), lambda i: (i, 0)),
            pl.BlockSpec((DF, DG), lambda i: (0, 0)),
            degp_spec, degp_spec,
        ],
        out_specs=[
            pl.BlockSpec((RBA, DG), lambda i: (i, 0)),
            deg_spec, deg_spec,
        ],
        out_shape=[
            jax.ShapeDtypeStruct((N, DG), jnp.float32),
            jax.ShapeDtypeStruct((N, 1), jnp.float32),
            jax.ShapeDtypeStruct((N, 1), jnp.float32),
        ],
    )(features, W1, indeg_p.reshape(NC, NPAD, 1),
      outdeg_p.reshape(NC, NPAD, 1))


def _tcb_body(agg_p, y, dinv, dinvl, b1, w1, bb1, w2, bb2,
              u_o, cs_o, sts_o):
    di = dinv[...]
    agg = agg_p[0] + agg_p[1]
    nf2 = di * (agg + y[...]) + b1[...]
    h1 = jnp.tanh(jnp.dot(nf2, w1[...], preferred_element_type=jnp.float32)
                  + bb1[...])
    lg = jnp.dot(h1, w2[...], preferred_element_type=jnp.float32) + bb2[...]
    m = jnp.max(lg, axis=1, keepdims=True)
    ex = jnp.exp(lg - m)
    S = ex / jnp.sum(ex, axis=1, keepdims=True)
    u_o[...] = dinvl[...] * S

    @pl.when(pl.program_id(0) == 0)
    def _():
        cs_o[...] = jnp.zeros_like(cs_o)
        sts_o[...] = jnp.zeros_like(sts_o)

    cs_o[...] += jnp.sum(nf2, axis=0, keepdims=True)
    sts_o[...] += lax.dot_general(S, S, (((0,), (0,)), ((), ())),
                                  preferred_element_type=jnp.float32)


def _tc_b(agg_p, y, dinv, dinvl, b1, fc1_W, fc1_b, fc2_W, fc2_b):
    g = N // RB
    deg_spec = pl.BlockSpec((RB, 1), lambda i: (i, 0))
    return pl.pallas_call(
        _tcb_body,
        grid=(g,),
        in_specs=[
            pl.BlockSpec((NC, RB, DG), lambda i: (0, i, 0)),
            pl.BlockSpec((RB, DG), lambda i: (i, 0)),
            deg_spec, deg_spec,
            pl.BlockSpec((1, DG), lambda i: (0, 0)),
            pl.BlockSpec((DG, D1), lambda i: (0, 0)),
            pl.BlockSpec((1, D1), lambda i: (0, 0)),
            pl.BlockSpec((D1, D2), lambda i: (0, 0)),
            pl.BlockSpec((1, D2), lambda i: (0, 0)),
        ],
        out_specs=[
            pl.BlockSpec((RB, D2), lambda i: (i, 0)),
            pl.BlockSpec((1, DG), lambda i: (0, 0)),
            pl.BlockSpec((D2, D2), lambda i: (0, 0)),
        ],
        out_shape=[
            jax.ShapeDtypeStruct((N, D2), jnp.float32),
            jax.ShapeDtypeStruct((1, DG), jnp.float32),
            jax.ShapeDtypeStruct((D2, D2), jnp.float32),
        ],
    )(agg_p, y, dinv, dinvl, b1, fc1_W, fc1_b, fc2_W, fc2_b)


def _tcc_body(u, t_p, sts, colsum, ge_o, pp_o, utt_o):
    t = t_p[0] + t_p[1]

    @pl.when(pl.program_id(0) == 0)
    def _():
        utt_o[...] = jnp.zeros_like(utt_o)

    utt_o[...] += lax.dot_general(u[...], t, (((0,), (0,)), ((), ())),
                                  preferred_element_type=jnp.float32)

    @pl.when(pl.program_id(0) == N // RB - 1)
    def _():
        new_adj = sts[...] - utt_o[...]
        row_norm = jnp.sum(jnp.abs(new_adj), axis=1, keepdims=True)
        nrm = new_adj / jnp.maximum(row_norm, 1e-12)
        r_i = lax.broadcasted_iota(jnp.int32, (D2, D2), 0)
        c_i = lax.broadcasted_iota(jnp.int32, (D2, D2), 1)
        eye = jnp.where(r_i == c_i, 1.0, 0.0).astype(jnp.float32)
        d = jnp.sum(nrm * eye, axis=0, keepdims=True)
        pp = jnp.mean((d - eye) ** 2)
        pp_o[...] = jnp.reshape(pp, (1, 1))
        ge_o[...] = colsum[...] * jnp.float32(1.0 / D2)


def _tc_c(U, t_p, StS, colsum):
    g = N // RB
    return pl.pallas_call(
        _tcc_body,
        grid=(g,),
        in_specs=[
            pl.BlockSpec((RB, D2), lambda i: (i, 0)),
            pl.BlockSpec((NC, RB, D2), lambda i: (0, i, 0)),
            pl.BlockSpec((D2, D2), lambda i: (0, 0)),
            pl.BlockSpec((1, DG), lambda i: (0, 0)),
        ],
        out_specs=[
            pl.BlockSpec((1, DG), lambda i: (0, 0)),
            pl.BlockSpec((1, 1), lambda i: (0, 0)),
            pl.BlockSpec((D2, D2), lambda i: (0, 0)),
        ],
        out_shape=[
            jax.ShapeDtypeStruct((1, DG), jnp.float32),
            jax.ShapeDtypeStruct((1, 1), jnp.float32),
            jax.ShapeDtypeStruct((D2, D2), jnp.float32),
        ],
    )(U, t_p, StS, colsum)


def kernel(features, edges, W1, b1, fc1_W, fc1_b, fc2_W, fc2_b):
    e2 = edges.reshape(2, NW, EPW)

    indeg_p, outdeg_p = _sc_degrees(e2)

    Y, dinv, dinvl = _tc_a(features, W1, indeg_p, outdeg_p)

    agg_p = _sc_agg(e2, Y)

    U, colsum, StS = _tc_b(
        agg_p, Y, dinv, dinvl,
        b1.reshape(1, DG), fc1_W, fc1_b.reshape(1, D1),
        fc2_W, fc2_b.reshape(1, D2))

    t_p = _sc_lap(e2, U)

    graph_embedding, pp, _ = _tc_c(U, t_p, StS, colsum)
    return (graph_embedding, pp.reshape(()))
